# Initial kernel scaffold; baseline (speedup 1.0000x reference)
import jax
import jax.numpy as jnp
from jax.experimental import pallas as pl


def kernel(x, edge_index, batch, W0, b0, W1, b1, Wl, bl):
    raise NotImplementedError("write your pallas kernel here")


# speedup vs baseline: 15.9744x; 15.9744x over previous
import functools

import jax
import jax.numpy as jnp
from jax import lax
from jax.experimental import pallas as pl
from jax.experimental.pallas import tpu as pltpu
from jax.experimental.pallas import tpu_sc as plsc

N = 100000
E = 3200000
H = 16
G = 1000
D_IN = 5

CH = 128
N_CH_FULL = N // CH
N_TAIL = N - N_CH_FULL * CH
N_PAD = (N_CH_FULL + 1) * CH
NP_CH = N_PAD // CH
E_CH = E // CH
G_PAD = 1024
GP_CH = G_PAD // CH

NC = 2
NS = 16
E_CH_SC = E_CH // NC

_mesh = plsc.VectorSubcoreMesh(core_axis_name="c", subcore_axis_name="s")
_sc_params = pltpu.CompilerParams(use_tc_tiling_on_sc=False)


def _zero_rows(zrows):
  z16 = jnp.zeros((16,), jnp.float32)
  for j in range(CH):
    zrows[j] = z16


@functools.partial(
    pl.kernel,
    out_type=[
        jax.ShapeDtypeStruct((NC, N_PAD), jnp.float32),
        jax.ShapeDtypeStruct((NC, G_PAD), jnp.float32),
    ],
    mesh=_mesh,
    compiler_params=_sc_params,
    scratch_types=[
        pltpu.VMEM_SHARED((N_PAD,), jnp.float32),
        pltpu.VMEM_SHARED((G_PAD,), jnp.float32),
        pltpu.VMEM((CH,), jnp.int32),
        pltpu.VMEM((CH,), jnp.float32),
        pltpu.VMEM((CH,), jnp.float32),
        pltpu.VMEM((N_TAIL,), jnp.int32),
        pltpu.VMEM((N_TAIL,), jnp.float32),
    ],
)
def _sc_count(dst_hbm, batch_hbm, cnt_hbm, gcnt_hbm,
              accd, accg, idxb, ones, zb, idx32, ones32):
  c = lax.axis_index("c")
  s = lax.axis_index("s")
  w = s * NC + c

  one16 = jnp.full((16,), 1.0, jnp.float32)
  zero16 = jnp.zeros((16,), jnp.float32)
  for j in range(CH // 16):
    ones[pl.ds(j * 16, 16)] = one16
    zb[pl.ds(j * 16, 16)] = zero16
  for j in range(N_TAIL // 16):
    ones32[pl.ds(j * 16, 16)] = one16

  @pl.loop(0, (NP_CH + NS - 1) // NS)
  def _(i):
    ch = s + i * NS

    @pl.when(ch < NP_CH)
    def _():
      pltpu.sync_copy(zb, accd.at[pl.ds(ch * CH, CH)])

  @pl.when(s < GP_CH)
  def _():
    pltpu.sync_copy(zb, accg.at[pl.ds(s * CH, CH)])

  plsc.subcore_barrier()

  @pl.loop(0, (E_CH_SC + NS - 1) // NS)
  def _(i):
    ch = s + i * NS

    @pl.when(ch < E_CH_SC)
    def _():
      base = (c * E_CH_SC + ch) * CH
      pltpu.sync_copy(dst_hbm.at[pl.ds(base, CH)], idxb)
      pltpu.sync_copy(ones, accd.at[idxb], add=True)

  @pl.loop(0, (N_CH_FULL + NC * NS - 1) // (NC * NS))
  def _(i):
    ch = w + i * NC * NS

    @pl.when(ch < N_CH_FULL)
    def _():
      pltpu.sync_copy(batch_hbm.at[pl.ds(ch * CH, CH)], idxb)
      pltpu.sync_copy(ones, accg.at[idxb], add=True)

  @pl.when(w == 13)
  def _():
    pltpu.sync_copy(batch_hbm.at[pl.ds(N_CH_FULL * CH, N_TAIL)], idx32)
    pltpu.sync_copy(ones32, accg.at[idx32], add=True)

  plsc.subcore_barrier()

  @pl.loop(0, (NP_CH + NS - 1) // NS)
  def _(i):
    ch = s + i * NS

    @pl.when(ch < NP_CH)
    def _():
      pltpu.sync_copy(accd.at[pl.ds(ch * CH, CH)],
                      cnt_hbm.at[c, pl.ds(ch * CH, CH)])

  @pl.when(s < GP_CH)
  def _():
    pltpu.sync_copy(accg.at[pl.ds(s * CH, CH)],
                    gcnt_hbm.at[c, pl.ds(s * CH, CH)])


@functools.partial(
    pl.kernel,
    out_type=jax.ShapeDtypeStruct((NC, N_PAD, H), jnp.float32),
    mesh=_mesh,
    compiler_params=_sc_params,
    scratch_types=[
        pltpu.VMEM_SHARED((N_PAD, H), jnp.float32),
        pltpu.VMEM((CH,), jnp.int32),
        pltpu.VMEM((CH,), jnp.int32),
        pltpu.VMEM((CH, H), jnp.float32),
        pltpu.VMEM((CH, H), jnp.float32),
        pltpu.SemaphoreType.DMA,
    ],
)
def _sc_scatter(u_hbm, src_hbm, dst_hbm, out_hbm,
                acc, idxs, idxd, rows, zrows, sem):
  c = lax.axis_index("c")
  s = lax.axis_index("s")

  _zero_rows(zrows)

  @pl.loop(0, (NP_CH + NS - 1) // NS)
  def _(i):
    ch = s + i * NS

    @pl.when(ch < NP_CH)
    def _():
      pltpu.sync_copy(zrows, acc.at[pl.ds(ch * CH, CH)])

  plsc.subcore_barrier()

  @pl.loop(0, (E_CH_SC + NS - 1) // NS)
  def _(i):
    ch = s + i * NS

    @pl.when(ch < E_CH_SC)
    def _():
      base = (c * E_CH_SC + ch) * CH
      pltpu.sync_copy(src_hbm.at[pl.ds(base, CH)], idxs)
      pltpu.sync_copy(dst_hbm.at[pl.ds(base, CH)], idxd)
      pltpu.async_copy(u_hbm.at[idxs], rows, sem).wait()
      pltpu.sync_copy(rows, acc.at[idxd], add=True)

  plsc.subcore_barrier()

  @pl.loop(0, (NP_CH + NS - 1) // NS)
  def _(i):
    ch = s + i * NS

    @pl.when(ch < NP_CH)
    def _():
      pltpu.sync_copy(acc.at[pl.ds(ch * CH, CH)],
                      out_hbm.at[c, pl.ds(ch * CH, CH)])


@functools.partial(
    pl.kernel,
    out_type=jax.ShapeDtypeStruct((NC, G_PAD, H), jnp.float32),
    mesh=_mesh,
    compiler_params=_sc_params,
    scratch_types=[
        pltpu.VMEM_SHARED((G_PAD, H), jnp.float32),
        pltpu.VMEM((CH,), jnp.int32),
        pltpu.VMEM((CH, H), jnp.float32),
        pltpu.VMEM((CH, H), jnp.float32),
        pltpu.VMEM((N_TAIL,), jnp.int32),
        pltpu.VMEM((N_TAIL, H), jnp.float32),
    ],
)
def _sc_pool(h_hbm, batch_hbm, out_hbm,
             accp, idxb, rows, zrows, idx32, rows32):
  c = lax.axis_index("c")
  s = lax.axis_index("s")
  w = s * NC + c

  _zero_rows(zrows)

  @pl.when(s < GP_CH)
  def _():
    pltpu.sync_copy(zrows, accp.at[pl.ds(s * CH, CH)])

  plsc.subcore_barrier()

  @pl.loop(0, (N_CH_FULL + NC * NS - 1) // (NC * NS))
  def _(i):
    ch = w + i * NC * NS

    @pl.when(ch < N_CH_FULL)
    def _():
      pltpu.sync_copy(batch_hbm.at[pl.ds(ch * CH, CH)], idxb)
      pltpu.sync_copy(h_hbm.at[pl.ds(ch * CH, CH)], rows)
      pltpu.sync_copy(rows, accp.at[idxb], add=True)

  @pl.when(w == 13)
  def _():
    pltpu.sync_copy(batch_hbm.at[pl.ds(N_CH_FULL * CH, N_TAIL)], idx32)
    pltpu.sync_copy(h_hbm.at[pl.ds(N_CH_FULL * CH, N_TAIL)], rows32)
    pltpu.sync_copy(rows32, accp.at[idx32], add=True)

  plsc.subcore_barrier()

  @pl.when(s < GP_CH)
  def _():
    pltpu.sync_copy(accp.at[pl.ds(s * CH, CH)],
                    out_hbm.at[c, pl.ds(s * CH, CH)])


_BN = 128
_GRID = NP_CH


def _tc_a_body(x_ref, c0_ref, c1_ref, w_ref, u_ref, d_ref):
  deg = c0_ref[...] + c1_ref[...] + 1.0
  dinv = lax.rsqrt(deg)
  d16 = jnp.broadcast_to(dinv, (_BN, H))
  u_ref[...] = jnp.dot(x_ref[...], w_ref[...],
                       preferred_element_type=jnp.float32) * d16
  d_ref[...] = d16


def _tc_a(x, c0, c1, W0):
  return pl.pallas_call(
      _tc_a_body,
      grid=(_GRID,),
      in_specs=[
          pl.BlockSpec((_BN, D_IN), lambda i: (i, 0)),
          pl.BlockSpec((_BN, 1), lambda i: (i, 0)),
          pl.BlockSpec((_BN, 1), lambda i: (i, 0)),
          pl.BlockSpec((D_IN, H), lambda i: (0, 0)),
      ],
      out_specs=[
          pl.BlockSpec((_BN, H), lambda i: (i, 0)),
          pl.BlockSpec((_BN, H), lambda i: (i, 0)),
      ],
      out_shape=[
          jax.ShapeDtypeStruct((N, H), jnp.float32),
          jax.ShapeDtypeStruct((N, H), jnp.float32),
      ],
  )(x, c0, c1, W0)


def _tc_b_body(sa_ref, sb_ref, u_ref, d_ref, b_ref, w_ref, u1_ref):
  h1 = (sa_ref[...] + sb_ref[...] + u_ref[...]) * d_ref[...] + b_ref[...]
  h1 = jnp.maximum(h1, 0.0)
  u1_ref[...] = jnp.dot(h1, w_ref[...],
                        preferred_element_type=jnp.float32) * d_ref[...]


def _tc_b(sa, sb, u0, d16, b0, W1):
  return pl.pallas_call(
      _tc_b_body,
      grid=(_GRID,),
      in_specs=[
          pl.BlockSpec((_BN, H), lambda i: (i, 0)),
          pl.BlockSpec((_BN, H), lambda i: (i, 0)),
          pl.BlockSpec((_BN, H), lambda i: (i, 0)),
          pl.BlockSpec((_BN, H), lambda i: (i, 0)),
          pl.BlockSpec((1, H), lambda i: (0, 0)),
          pl.BlockSpec((H, H), lambda i: (0, 0)),
      ],
      out_specs=pl.BlockSpec((_BN, H), lambda i: (i, 0)),
      out_shape=jax.ShapeDtypeStruct((N, H), jnp.float32),
  )(sa, sb, u0, d16, b0, W1)


def _tc_c_body(sa_ref, sb_ref, u_ref, d_ref, b_ref, h_ref):
  h_ref[...] = (sa_ref[...] + sb_ref[...] + u_ref[...]) * d_ref[...] \
      + b_ref[...]


def _tc_c(sa, sb, u1, d16, b1):
  return pl.pallas_call(
      _tc_c_body,
      grid=(_GRID,),
      in_specs=[
          pl.BlockSpec((_BN, H), lambda i: (i, 0)),
          pl.BlockSpec((_BN, H), lambda i: (i, 0)),
          pl.BlockSpec((_BN, H), lambda i: (i, 0)),
          pl.BlockSpec((_BN, H), lambda i: (i, 0)),
          pl.BlockSpec((1, H), lambda i: (0, 0)),
      ],
      out_specs=pl.BlockSpec((_BN, H), lambda i: (i, 0)),
      out_shape=jax.ShapeDtypeStruct((N, H), jnp.float32),
  )(sa, sb, u1, d16, b1)


def _tc_d_body(pa_ref, pb_ref, ga_ref, gb_ref, wl_ref, bl_ref, o_ref):
  cnt = jnp.maximum(ga_ref[...] + gb_ref[...], 1.0)
  p = (pa_ref[...] + pb_ref[...]) / cnt
  o_ref[...] = jax.nn.sigmoid(
      jnp.dot(p, wl_ref[...], preferred_element_type=jnp.float32)
      + bl_ref[...])


def _tc_d(pa, pb, ga, gb, Wl, bl):
  return pl.pallas_call(
      _tc_d_body,
      out_shape=jax.ShapeDtypeStruct((G_PAD, 1), jnp.float32),
  )(pa, pb, ga, gb, Wl, bl)


def kernel(x, edge_index, batch, W0, b0, W1, b1, Wl, bl):
  src = edge_index[0]
  dst = edge_index[1]

  cnt2, gcnt2 = _sc_count(dst, batch)

  u0, d16 = _tc_a(x, cnt2[0].reshape(N_PAD, 1), cnt2[1].reshape(N_PAD, 1),
                  W0)

  s0 = _sc_scatter(u0, src, dst)
  u1 = _tc_b(s0[0, :N], s0[1, :N], u0, d16, b0.reshape(1, H), W1)

  s1 = _sc_scatter(u1, src, dst)
  h2 = _tc_c(s1[0, :N], s1[1, :N], u1, d16, b1.reshape(1, H))

  p2 = _sc_pool(h2, batch)

  out = _tc_d(p2[0], p2[1],
              gcnt2[0].reshape(G_PAD, 1), gcnt2[1].reshape(G_PAD, 1),
              Wl, bl.reshape(1, 1))
  return out[:G]


# speedup vs baseline: 35.9009x; 2.2474x over previous
import functools

import jax
import jax.numpy as jnp
from jax import lax
from jax.experimental import pallas as pl
from jax.experimental.pallas import tpu as pltpu
from jax.experimental.pallas import tpu_sc as plsc

N = 100000
E = 3200000
H = 16
G = 1000
D_IN = 5

CH = 128
N_CH_FULL = N // CH
N_TAIL = N - N_CH_FULL * CH
N_PAD = (N_CH_FULL + 1) * CH
NP_CH = N_PAD // CH
E_CH = E // CH
G_PAD = 1024
GP_CH = G_PAD // CH

NC = 2
NS = 16
E_CH_SC = E_CH // NC

KG = 8
NG_TOT = E_CH // KG
NG_MAX = (NG_TOT + NC * NS - 1) // (NC * NS)
NG_REM = NG_TOT - (NG_MAX - 1) * NC * NS

_mesh = plsc.VectorSubcoreMesh(core_axis_name="c", subcore_axis_name="s")
_sc_params = pltpu.CompilerParams(use_tc_tiling_on_sc=False)


def _zero_rows(zrows):
  z16 = jnp.zeros((16,), jnp.float32)
  for j in range(CH):
    zrows[j] = z16


@functools.partial(
    pl.kernel,
    out_type=[
        jax.ShapeDtypeStruct((NC, N_PAD), jnp.float32),
        jax.ShapeDtypeStruct((NC, G_PAD), jnp.float32),
    ],
    mesh=_mesh,
    compiler_params=_sc_params,
    scratch_types=[
        pltpu.VMEM_SHARED((N_PAD,), jnp.float32),
        pltpu.VMEM_SHARED((G_PAD,), jnp.float32),
        pltpu.VMEM((2, KG, CH), jnp.int32),
        pltpu.VMEM((CH,), jnp.int32),
        pltpu.VMEM((CH,), jnp.float32),
        pltpu.VMEM((CH,), jnp.float32),
        pltpu.VMEM((N_TAIL,), jnp.int32),
        pltpu.VMEM((N_TAIL,), jnp.float32),
        pltpu.SemaphoreType.DMA,
        pltpu.SemaphoreType.DMA,
        pltpu.SemaphoreType.DMA,
    ],
)
def _sc_count(dst_hbm, batch_hbm, cnt_hbm, gcnt_hbm,
              accd, accg, idbuf, idxb, ones, zb, idx32, ones32,
              semi, sems, semz):
  c = lax.axis_index("c")
  s = lax.axis_index("s")
  w = s * NC + c
  ng = jnp.where(w < NG_REM, NG_MAX, NG_MAX - 1)

  one16 = jnp.full((16,), 1.0, jnp.float32)
  zero16 = jnp.zeros((16,), jnp.float32)
  for j in range(CH // 16):
    ones[pl.ds(j * 16, 16)] = one16
    zb[pl.ds(j * 16, 16)] = zero16
  for j in range(N_TAIL // 16):
    ones32[pl.ds(j * 16, 16)] = one16

  pltpu.make_async_copy(dst_hbm.at[pl.ds(w * KG, KG)], idbuf.at[0],
                        semi).start()

  @pl.loop(0, (NP_CH + NS - 1) // NS)
  def _(i):
    ch = s + i * NS

    @pl.when(ch < NP_CH)
    def _():
      pltpu.make_async_copy(zb, accd.at[pl.ds(ch * CH, CH)], semz).start()

  @pl.when(s < GP_CH)
  def _():
    pltpu.make_async_copy(zb, accg.at[pl.ds(s * CH, CH)], semz).start()

  @pl.loop(0, (NP_CH + NS - 1) // NS)
  def _(i):
    ch = s + i * NS

    @pl.when(ch < NP_CH)
    def _():
      pltpu.make_async_copy(zb, accd.at[pl.ds(ch * CH, CH)], semz).wait()

  @pl.when(s < GP_CH)
  def _():
    pltpu.make_async_copy(zb, accg.at[pl.ds(s * CH, CH)], semz).wait()

  plsc.subcore_barrier()

  @pl.loop(0, NG_MAX)
  def _(i):
    @pl.when(i < ng)
    def _():
      g = w + i * NC * NS
      b = i % 2
      pltpu.make_async_copy(dst_hbm.at[pl.ds(g * KG, KG)], idbuf.at[b],
                            semi).wait()

      @pl.when(i > 0)
      def _():
        for j in range(KG):
          pltpu.make_async_copy(ones, accd.at[idbuf.at[1 - b, j]],
                                sems).wait()

      @pl.when(i + 1 < ng)
      def _():
        g2 = w + (i + 1) * NC * NS
        pltpu.make_async_copy(dst_hbm.at[pl.ds(g2 * KG, KG)],
                              idbuf.at[1 - b], semi).start()

      for j in range(KG):
        pltpu.make_async_copy(ones, accd.at[idbuf.at[b, j]],
                              sems).start(add=True)

  for j in range(KG):
    pltpu.make_async_copy(ones, accd.at[idbuf.at[0, j]], sems).wait()

  @pl.loop(0, (N_CH_FULL + NC * NS - 1) // (NC * NS))
  def _(i):
    ch = w + i * NC * NS

    @pl.when(ch < N_CH_FULL)
    def _():
      pltpu.sync_copy(batch_hbm.at[pl.ds(ch * CH, CH)], idxb)
      pltpu.sync_copy(ones, accg.at[idxb], add=True)

  @pl.when(w == 13)
  def _():
    pltpu.sync_copy(batch_hbm.at[pl.ds(N_CH_FULL * CH, N_TAIL)], idx32)
    pltpu.sync_copy(ones32, accg.at[idx32], add=True)

  plsc.subcore_barrier()

  @pl.loop(0, (NP_CH + NS - 1) // NS)
  def _(i):
    ch = s + i * NS

    @pl.when(ch < NP_CH)
    def _():
      pltpu.make_async_copy(accd.at[pl.ds(ch * CH, CH)],
                            cnt_hbm.at[c, pl.ds(ch * CH, CH)], semz).start()

  @pl.when(s < GP_CH)
  def _():
    pltpu.make_async_copy(accg.at[pl.ds(s * CH, CH)],
                          gcnt_hbm.at[c, pl.ds(s * CH, CH)], semz).start()

  @pl.loop(0, (NP_CH + NS - 1) // NS)
  def _(i):
    ch = s + i * NS

    @pl.when(ch < NP_CH)
    def _():
      pltpu.make_async_copy(accd.at[pl.ds(ch * CH, CH)],
                            cnt_hbm.at[c, pl.ds(ch * CH, CH)], semz).wait()

  @pl.when(s < GP_CH)
  def _():
    pltpu.make_async_copy(accg.at[pl.ds(s * CH, CH)],
                          gcnt_hbm.at[c, pl.ds(s * CH, CH)], semz).wait()


@functools.partial(
    pl.kernel,
    out_type=jax.ShapeDtypeStruct((NC, N_PAD, H), jnp.float32),
    mesh=_mesh,
    compiler_params=_sc_params,
    scratch_types=[
        pltpu.VMEM_SHARED((N_PAD, H), jnp.float32),
        pltpu.VMEM((2, KG, CH), jnp.int32),
        pltpu.VMEM((2, KG, CH), jnp.int32),
        pltpu.VMEM((KG, CH, H), jnp.float32),
        pltpu.VMEM((CH, H), jnp.float32),
        pltpu.SemaphoreType.DMA,
        pltpu.SemaphoreType.DMA,
        pltpu.SemaphoreType.DMA((KG,)),
        pltpu.SemaphoreType.DMA,
        pltpu.SemaphoreType.DMA,
    ],
)
def _sc_scatter(u_hbm, src_hbm, dst_hbm, out_hbm,
                acc, isbuf, idbuf, rows, zrows,
                semis, semid, semg, sems, semz):
  c = lax.axis_index("c")
  s = lax.axis_index("s")
  w = s * NC + c
  ng = jnp.where(w < NG_REM, NG_MAX, NG_MAX - 1)

  _zero_rows(zrows)

  pltpu.make_async_copy(src_hbm.at[pl.ds(w * KG, KG)], isbuf.at[0],
                        semis).start()
  pltpu.make_async_copy(dst_hbm.at[pl.ds(w * KG, KG)], idbuf.at[0],
                        semid).start()

  @pl.loop(0, (NP_CH + NS - 1) // NS)
  def _(i):
    ch = s + i * NS

    @pl.when(ch < NP_CH)
    def _():
      pltpu.make_async_copy(zrows, acc.at[pl.ds(ch * CH, CH)], semz).start()

  @pl.loop(0, (NP_CH + NS - 1) // NS)
  def _(i):
    ch = s + i * NS

    @pl.when(ch < NP_CH)
    def _():
      pltpu.make_async_copy(zrows, acc.at[pl.ds(ch * CH, CH)], semz).wait()

  plsc.subcore_barrier()

  @pl.loop(0, NG_MAX)
  def _(i):
    @pl.when(i < ng)
    def _():
      g = w + i * NC * NS
      b = i % 2
      pltpu.make_async_copy(src_hbm.at[pl.ds(g * KG, KG)], isbuf.at[b],
                            semis).wait()
      pltpu.make_async_copy(dst_hbm.at[pl.ds(g * KG, KG)], idbuf.at[b],
                            semid).wait()

      @pl.when(i > 0)
      def _():
        for j in range(KG):
          pltpu.make_async_copy(rows.at[j], acc.at[idbuf.at[1 - b, j]],
                                sems).wait()

      @pl.when(i + 1 < ng)
      def _():
        g2 = w + (i + 1) * NC * NS
        pltpu.make_async_copy(src_hbm.at[pl.ds(g2 * KG, KG)],
                              isbuf.at[1 - b], semis).start()
        pltpu.make_async_copy(dst_hbm.at[pl.ds(g2 * KG, KG)],
                              idbuf.at[1 - b], semid).start()

      for j in range(KG):
        pltpu.make_async_copy(u_hbm.at[isbuf.at[b, j]], rows.at[j],
                              semg.at[j]).start()
      for j in range(KG):
        pltpu.make_async_copy(u_hbm.at[isbuf.at[b, j]], rows.at[j],
                              semg.at[j]).wait()
        pltpu.make_async_copy(rows.at[j], acc.at[idbuf.at[b, j]],
                              sems).start(add=True)

  for j in range(KG):
    pltpu.make_async_copy(rows.at[j], acc.at[idbuf.at[0, j]], sems).wait()

  plsc.subcore_barrier()

  @pl.loop(0, (NP_CH + NS - 1) // NS)
  def _(i):
    ch = s + i * NS

    @pl.when(ch < NP_CH)
    def _():
      pltpu.make_async_copy(acc.at[pl.ds(ch * CH, CH)],
                            out_hbm.at[c, pl.ds(ch * CH, CH)], semz).start()

  @pl.loop(0, (NP_CH + NS - 1) // NS)
  def _(i):
    ch = s + i * NS

    @pl.when(ch < NP_CH)
    def _():
      pltpu.make_async_copy(acc.at[pl.ds(ch * CH, CH)],
                            out_hbm.at[c, pl.ds(ch * CH, CH)], semz).wait()


@functools.partial(
    pl.kernel,
    out_type=jax.ShapeDtypeStruct((NC, G_PAD, H), jnp.float32),
    mesh=_mesh,
    compiler_params=_sc_params,
    scratch_types=[
        pltpu.VMEM_SHARED((G_PAD, H), jnp.float32),
        pltpu.VMEM((CH,), jnp.int32),
        pltpu.VMEM((CH, H), jnp.float32),
        pltpu.VMEM((CH, H), jnp.float32),
        pltpu.VMEM((N_TAIL,), jnp.int32),
        pltpu.VMEM((N_TAIL, H), jnp.float32),
    ],
)
def _sc_pool(h_hbm, batch_hbm, out_hbm,
             accp, idxb, rows, zrows, idx32, rows32):
  c = lax.axis_index("c")
  s = lax.axis_index("s")
  w = s * NC + c

  _zero_rows(zrows)

  @pl.when(s < GP_CH)
  def _():
    pltpu.sync_copy(zrows, accp.at[pl.ds(s * CH, CH)])

  plsc.subcore_barrier()

  @pl.loop(0, (N_CH_FULL + NC * NS - 1) // (NC * NS))
  def _(i):
    ch = w + i * NC * NS

    @pl.when(ch < N_CH_FULL)
    def _():
      pltpu.sync_copy(batch_hbm.at[pl.ds(ch * CH, CH)], idxb)
      pltpu.sync_copy(h_hbm.at[pl.ds(ch * CH, CH)], rows)
      pltpu.sync_copy(rows, accp.at[idxb], add=True)

  @pl.when(w == 13)
  def _():
    pltpu.sync_copy(batch_hbm.at[pl.ds(N_CH_FULL * CH, N_TAIL)], idx32)
    pltpu.sync_copy(h_hbm.at[pl.ds(N_CH_FULL * CH, N_TAIL)], rows32)
    pltpu.sync_copy(rows32, accp.at[idx32], add=True)

  plsc.subcore_barrier()

  @pl.when(s < GP_CH)
  def _():
    pltpu.sync_copy(accp.at[pl.ds(s * CH, CH)],
                    out_hbm.at[c, pl.ds(s * CH, CH)])


_BN = 128
_GRID = NP_CH


def _tc_a_body(x_ref, c0_ref, c1_ref, w_ref, u_ref, d_ref):
  deg = c0_ref[...] + c1_ref[...] + 1.0
  dinv = lax.rsqrt(deg)
  d16 = jnp.broadcast_to(dinv, (_BN, H))
  u_ref[...] = jnp.dot(x_ref[...], w_ref[...],
                       preferred_element_type=jnp.float32) * d16
  d_ref[...] = d16


def _tc_a(x, c0, c1, W0):
  return pl.pallas_call(
      _tc_a_body,
      grid=(_GRID,),
      in_specs=[
          pl.BlockSpec((_BN, D_IN), lambda i: (i, 0)),
          pl.BlockSpec((_BN, 1), lambda i: (i, 0)),
          pl.BlockSpec((_BN, 1), lambda i: (i, 0)),
          pl.BlockSpec((D_IN, H), lambda i: (0, 0)),
      ],
      out_specs=[
          pl.BlockSpec((_BN, H), lambda i: (i, 0)),
          pl.BlockSpec((_BN, H), lambda i: (i, 0)),
      ],
      out_shape=[
          jax.ShapeDtypeStruct((N, H), jnp.float32),
          jax.ShapeDtypeStruct((N, H), jnp.float32),
      ],
  )(x, c0, c1, W0)


def _tc_b_body(sa_ref, sb_ref, u_ref, d_ref, b_ref, w_ref, u1_ref):
  h1 = (sa_ref[...] + sb_ref[...] + u_ref[...]) * d_ref[...] + b_ref[...]
  h1 = jnp.maximum(h1, 0.0)
  u1_ref[...] = jnp.dot(h1, w_ref[...],
                        preferred_element_type=jnp.float32) * d_ref[...]


def _tc_b(sa, sb, u0, d16, b0, W1):
  return pl.pallas_call(
      _tc_b_body,
      grid=(_GRID,),
      in_specs=[
          pl.BlockSpec((_BN, H), lambda i: (i, 0)),
          pl.BlockSpec((_BN, H), lambda i: (i, 0)),
          pl.BlockSpec((_BN, H), lambda i: (i, 0)),
          pl.BlockSpec((_BN, H), lambda i: (i, 0)),
          pl.BlockSpec((1, H), lambda i: (0, 0)),
          pl.BlockSpec((H, H), lambda i: (0, 0)),
      ],
      out_specs=pl.BlockSpec((_BN, H), lambda i: (i, 0)),
      out_shape=jax.ShapeDtypeStruct((N, H), jnp.float32),
  )(sa, sb, u0, d16, b0, W1)


def _tc_c_body(sa_ref, sb_ref, u_ref, d_ref, b_ref, h_ref):
  h_ref[...] = (sa_ref[...] + sb_ref[...] + u_ref[...]) * d_ref[...] \
      + b_ref[...]


def _tc_c(sa, sb, u1, d16, b1):
  return pl.pallas_call(
      _tc_c_body,
      grid=(_GRID,),
      in_specs=[
          pl.BlockSpec((_BN, H), lambda i: (i, 0)),
          pl.BlockSpec((_BN, H), lambda i: (i, 0)),
          pl.BlockSpec((_BN, H), lambda i: (i, 0)),
          pl.BlockSpec((_BN, H), lambda i: (i, 0)),
          pl.BlockSpec((1, H), lambda i: (0, 0)),
      ],
      out_specs=pl.BlockSpec((_BN, H), lambda i: (i, 0)),
      out_shape=jax.ShapeDtypeStruct((N, H), jnp.float32),
  )(sa, sb, u1, d16, b1)


def _tc_d_body(pa_ref, pb_ref, ga_ref, gb_ref, wl_ref, bl_ref, o_ref):
  cnt = jnp.maximum(ga_ref[...] + gb_ref[...], 1.0)
  p = (pa_ref[...] + pb_ref[...]) / cnt
  o_ref[...] = jax.nn.sigmoid(
      jnp.dot(p, wl_ref[...], preferred_element_type=jnp.float32)
      + bl_ref[...])


def _tc_d(pa, pb, ga, gb, Wl, bl):
  return pl.pallas_call(
      _tc_d_body,
      out_shape=jax.ShapeDtypeStruct((G_PAD, 1), jnp.float32),
  )(pa, pb, ga, gb, Wl, bl)


def kernel(x, edge_index, batch, W0, b0, W1, b1, Wl, bl):
  src = edge_index[0].reshape(E_CH, CH)
  dst = edge_index[1].reshape(E_CH, CH)

  cnt2, gcnt2 = _sc_count(dst, batch)

  u0, d16 = _tc_a(x, cnt2[0].reshape(N_PAD, 1), cnt2[1].reshape(N_PAD, 1),
                  W0)

  s0 = _sc_scatter(u0, src, dst)
  u1 = _tc_b(s0[0, :N], s0[1, :N], u0, d16, b0.reshape(1, H), W1)

  s1 = _sc_scatter(u1, src, dst)
  h2 = _tc_c(s1[0, :N], s1[1, :N], u1, d16, b1.reshape(1, H))

  p2 = _sc_pool(h2, batch)

  out = _tc_d(p2[0], p2[1],
              gcnt2[0].reshape(G_PAD, 1), gcnt2[1].reshape(G_PAD, 1),
              Wl, bl.reshape(1, 1))
  return out[:G]


# speedup vs baseline: 77.5811x; 2.1610x over previous
import functools

import jax
import jax.numpy as jnp
from jax import lax
from jax.experimental import pallas as pl
from jax.experimental.pallas import tpu as pltpu
from jax.experimental.pallas import tpu_sc as plsc

N = 100000
E = 3200000
H = 16
G = 1000
D_IN = 5

CH = 128
N_CH_FULL = N // CH
N_TAIL = N - N_CH_FULL * CH
N_PAD = (N_CH_FULL + 1) * CH
NP_CH = N_PAD // CH
E_CH = E // CH
G_PAD = 1024
GP_CH = G_PAD // CH

NC = 2
NS = 16
E_CH_SC = E_CH // NC

KG = 8
NG_TOT = E_CH // KG
NG_MAX = (NG_TOT + NC * NS - 1) // (NC * NS)
NG_REM = NG_TOT - (NG_MAX - 1) * NC * NS

_mesh = plsc.VectorSubcoreMesh(core_axis_name="c", subcore_axis_name="s")
_sc_params = pltpu.CompilerParams(use_tc_tiling_on_sc=False)


def _zero_rows(zrows):
  z16 = jnp.zeros((16,), jnp.float32)
  for j in range(CH):
    zrows[j] = z16


@functools.partial(
    pl.kernel,
    out_type=[
        jax.ShapeDtypeStruct((NC, N_PAD), jnp.float32),
        jax.ShapeDtypeStruct((NC, G_PAD), jnp.float32),
    ],
    mesh=_mesh,
    compiler_params=_sc_params,
    scratch_types=[
        pltpu.VMEM_SHARED((N_PAD,), jnp.float32),
        pltpu.VMEM_SHARED((G_PAD,), jnp.float32),
        pltpu.VMEM((2, KG, CH), jnp.int32),
        pltpu.VMEM((CH,), jnp.int32),
        pltpu.VMEM((CH,), jnp.float32),
        pltpu.VMEM((CH,), jnp.float32),
        pltpu.VMEM((N_TAIL,), jnp.int32),
        pltpu.VMEM((N_TAIL,), jnp.float32),
        pltpu.SemaphoreType.DMA,
        pltpu.SemaphoreType.DMA,
        pltpu.SemaphoreType.DMA,
    ],
)
def _sc_count(dst_hbm, batch_hbm, cnt_hbm, gcnt_hbm,
              accd, accg, idbuf, idxb, ones, zb, idx32, ones32,
              semi, sems, semz):
  c = lax.axis_index("c")
  s = lax.axis_index("s")
  w = s * NC + c
  ng = jnp.where(w < NG_REM, NG_MAX, NG_MAX - 1)

  one16 = jnp.full((16,), 1.0, jnp.float32)
  zero16 = jnp.zeros((16,), jnp.float32)
  for j in range(CH // 16):
    ones[pl.ds(j * 16, 16)] = one16
    zb[pl.ds(j * 16, 16)] = zero16
  for j in range(N_TAIL // 16):
    ones32[pl.ds(j * 16, 16)] = one16

  pltpu.make_async_copy(dst_hbm.at[pl.ds(w * KG, KG)], idbuf.at[0],
                        semi).start()

  @pl.loop(0, (NP_CH + NS - 1) // NS)
  def _(i):
    ch = s + i * NS

    @pl.when(ch < NP_CH)
    def _():
      pltpu.make_async_copy(zb, accd.at[pl.ds(ch * CH, CH)], semz).start()

  @pl.when(s < GP_CH)
  def _():
    pltpu.make_async_copy(zb, accg.at[pl.ds(s * CH, CH)], semz).start()

  @pl.loop(0, (NP_CH + NS - 1) // NS)
  def _(i):
    ch = s + i * NS

    @pl.when(ch < NP_CH)
    def _():
      pltpu.make_async_copy(zb, accd.at[pl.ds(ch * CH, CH)], semz).wait()

  @pl.when(s < GP_CH)
  def _():
    pltpu.make_async_copy(zb, accg.at[pl.ds(s * CH, CH)], semz).wait()

  plsc.subcore_barrier()

  @pl.loop(0, NG_MAX)
  def _(i):
    @pl.when(i < ng)
    def _():
      g = w + i * NC * NS
      b = i % 2
      pltpu.make_async_copy(dst_hbm.at[pl.ds(g * KG, KG)], idbuf.at[b],
                            semi).wait()

      @pl.when(i > 0)
      def _():
        for j in range(KG):
          pltpu.make_async_copy(ones, accd.at[idbuf.at[1 - b, j]],
                                sems).wait()

      @pl.when(i + 1 < ng)
      def _():
        g2 = w + (i + 1) * NC * NS
        pltpu.make_async_copy(dst_hbm.at[pl.ds(g2 * KG, KG)],
                              idbuf.at[1 - b], semi).start()

      for j in range(KG):
        pltpu.make_async_copy(ones, accd.at[idbuf.at[b, j]],
                              sems).start(add=True)

  for j in range(KG):
    pltpu.make_async_copy(ones, accd.at[idbuf.at[0, j]], sems).wait()

  @pl.loop(0, (N_CH_FULL + NC * NS - 1) // (NC * NS))
  def _(i):
    ch = w + i * NC * NS

    @pl.when(ch < N_CH_FULL)
    def _():
      pltpu.sync_copy(batch_hbm.at[pl.ds(ch * CH, CH)], idxb)
      pltpu.sync_copy(ones, accg.at[idxb], add=True)

  @pl.when(w == 13)
  def _():
    pltpu.sync_copy(batch_hbm.at[pl.ds(N_CH_FULL * CH, N_TAIL)], idx32)
    pltpu.sync_copy(ones32, accg.at[idx32], add=True)

  plsc.subcore_barrier()

  @pl.loop(0, (NP_CH + NS - 1) // NS)
  def _(i):
    ch = s + i * NS

    @pl.when(ch < NP_CH)
    def _():
      pltpu.make_async_copy(accd.at[pl.ds(ch * CH, CH)],
                            cnt_hbm.at[c, pl.ds(ch * CH, CH)], semz).start()

  @pl.when(s < GP_CH)
  def _():
    pltpu.make_async_copy(accg.at[pl.ds(s * CH, CH)],
                          gcnt_hbm.at[c, pl.ds(s * CH, CH)], semz).start()

  @pl.loop(0, (NP_CH + NS - 1) // NS)
  def _(i):
    ch = s + i * NS

    @pl.when(ch < NP_CH)
    def _():
      pltpu.make_async_copy(accd.at[pl.ds(ch * CH, CH)],
                            cnt_hbm.at[c, pl.ds(ch * CH, CH)], semz).wait()

  @pl.when(s < GP_CH)
  def _():
    pltpu.make_async_copy(accg.at[pl.ds(s * CH, CH)],
                          gcnt_hbm.at[c, pl.ds(s * CH, CH)], semz).wait()


@functools.partial(
    pl.kernel,
    out_type=jax.ShapeDtypeStruct((NC, N_PAD, H), jnp.float32),
    mesh=_mesh,
    compiler_params=_sc_params,
    scratch_types=[
        pltpu.VMEM_SHARED((N_PAD, H), jnp.float32),
        pltpu.VMEM((2, KG, CH), jnp.int32),
        pltpu.VMEM((2, KG, CH), jnp.int32),
        pltpu.VMEM((KG, CH, H), jnp.float32),
        pltpu.VMEM((CH, H), jnp.float32),
        pltpu.SemaphoreType.DMA,
        pltpu.SemaphoreType.DMA,
        pltpu.SemaphoreType.DMA((KG,)),
        pltpu.SemaphoreType.DMA,
        pltpu.SemaphoreType.DMA,
    ],
)
def _sc_scatter(u_hbm, src_hbm, dst_hbm, out_hbm,
                acc, isbuf, idbuf, rows, zrows,
                semis, semid, semg, sems, semz):
  c = lax.axis_index("c")
  s = lax.axis_index("s")
  w = s * NC + c
  ng = jnp.where(w < NG_REM, NG_MAX, NG_MAX - 1)

  _zero_rows(zrows)

  pltpu.make_async_copy(src_hbm.at[pl.ds(w * KG, KG)], isbuf.at[0],
                        semis).start()
  pltpu.make_async_copy(dst_hbm.at[pl.ds(w * KG, KG)], idbuf.at[0],
                        semid).start()

  @pl.loop(0, (NP_CH + NS - 1) // NS)
  def _(i):
    ch = s + i * NS

    @pl.when(ch < NP_CH)
    def _():
      pltpu.make_async_copy(zrows, acc.at[pl.ds(ch * CH, CH)], semz).start()

  @pl.loop(0, (NP_CH + NS - 1) // NS)
  def _(i):
    ch = s + i * NS

    @pl.when(ch < NP_CH)
    def _():
      pltpu.make_async_copy(zrows, acc.at[pl.ds(ch * CH, CH)], semz).wait()

  plsc.subcore_barrier()

  @pl.loop(0, NG_MAX)
  def _(i):
    @pl.when(i < ng)
    def _():
      g = w + i * NC * NS
      b = i % 2
      pltpu.make_async_copy(src_hbm.at[pl.ds(g * KG, KG)], isbuf.at[b],
                            semis).wait()
      pltpu.make_async_copy(dst_hbm.at[pl.ds(g * KG, KG)], idbuf.at[b],
                            semid).wait()

      @pl.when(i > 0)
      def _():
        for j in range(KG):
          pltpu.make_async_copy(rows.at[j], acc.at[idbuf.at[1 - b, j]],
                                sems).wait()

      @pl.when(i + 1 < ng)
      def _():
        g2 = w + (i + 1) * NC * NS
        pltpu.make_async_copy(src_hbm.at[pl.ds(g2 * KG, KG)],
                              isbuf.at[1 - b], semis).start()
        pltpu.make_async_copy(dst_hbm.at[pl.ds(g2 * KG, KG)],
                              idbuf.at[1 - b], semid).start()

      for j in range(KG):
        pltpu.make_async_copy(u_hbm.at[isbuf.at[b, j]], rows.at[j],
                              semg.at[j]).start()
      for j in range(KG):
        pltpu.make_async_copy(u_hbm.at[isbuf.at[b, j]], rows.at[j],
                              semg.at[j]).wait()
        pltpu.make_async_copy(rows.at[j], acc.at[idbuf.at[b, j]],
                              sems).start(add=True)

  for j in range(KG):
    pltpu.make_async_copy(rows.at[j], acc.at[idbuf.at[0, j]], sems).wait()

  plsc.subcore_barrier()

  @pl.loop(0, (NP_CH + NS - 1) // NS)
  def _(i):
    ch = s + i * NS

    @pl.when(ch < NP_CH)
    def _():
      pltpu.make_async_copy(acc.at[pl.ds(ch * CH, CH)],
                            out_hbm.at[c, pl.ds(ch * CH, CH)], semz).start()

  @pl.loop(0, (NP_CH + NS - 1) // NS)
  def _(i):
    ch = s + i * NS

    @pl.when(ch < NP_CH)
    def _():
      pltpu.make_async_copy(acc.at[pl.ds(ch * CH, CH)],
                            out_hbm.at[c, pl.ds(ch * CH, CH)], semz).wait()


@functools.partial(
    pl.kernel,
    out_type=jax.ShapeDtypeStruct((NC, G_PAD, H), jnp.float32),
    mesh=_mesh,
    compiler_params=_sc_params,
    scratch_types=[
        pltpu.VMEM_SHARED((G_PAD, H), jnp.float32),
        pltpu.VMEM((CH,), jnp.int32),
        pltpu.VMEM((CH, H), jnp.float32),
        pltpu.VMEM((CH, H), jnp.float32),
        pltpu.VMEM((N_TAIL,), jnp.int32),
        pltpu.VMEM((N_TAIL, H), jnp.float32),
    ],
)
def _sc_pool(h_hbm, batch_hbm, out_hbm,
             accp, idxb, rows, zrows, idx32, rows32):
  c = lax.axis_index("c")
  s = lax.axis_index("s")
  w = s * NC + c

  _zero_rows(zrows)

  @pl.when(s < GP_CH)
  def _():
    pltpu.sync_copy(zrows, accp.at[pl.ds(s * CH, CH)])

  plsc.subcore_barrier()

  @pl.loop(0, (N_CH_FULL + NC * NS - 1) // (NC * NS))
  def _(i):
    ch = w + i * NC * NS

    @pl.when(ch < N_CH_FULL)
    def _():
      pltpu.sync_copy(batch_hbm.at[pl.ds(ch * CH, CH)], idxb)
      pltpu.sync_copy(h_hbm.at[pl.ds(ch * CH, CH)], rows)
      pltpu.sync_copy(rows, accp.at[idxb], add=True)

  @pl.when(w == 13)
  def _():
    pltpu.sync_copy(batch_hbm.at[pl.ds(N_CH_FULL * CH, N_TAIL)], idx32)
    pltpu.sync_copy(h_hbm.at[pl.ds(N_CH_FULL * CH, N_TAIL)], rows32)
    pltpu.sync_copy(rows32, accp.at[idx32], add=True)

  plsc.subcore_barrier()

  @pl.when(s < GP_CH)
  def _():
    pltpu.sync_copy(accp.at[pl.ds(s * CH, CH)],
                    out_hbm.at[c, pl.ds(s * CH, CH)])


_BN = 8192
_GRID = (N + _BN - 1) // _BN


def _tc_a_body(x_ref, c0_ref, c1_ref, w_ref, u_ref, d_ref):
  deg = c0_ref[...] + c1_ref[...] + 1.0
  dinv = lax.rsqrt(deg)
  d16 = jnp.broadcast_to(dinv, (_BN, H))
  u_ref[...] = jnp.dot(x_ref[...], w_ref[...],
                       preferred_element_type=jnp.float32) * d16
  d_ref[...] = d16


def _tc_a(x, c0, c1, W0):
  return pl.pallas_call(
      _tc_a_body,
      grid=(_GRID,),
      in_specs=[
          pl.BlockSpec((_BN, D_IN), lambda i: (i, 0)),
          pl.BlockSpec((_BN, 1), lambda i: (i, 0)),
          pl.BlockSpec((_BN, 1), lambda i: (i, 0)),
          pl.BlockSpec((D_IN, H), lambda i: (0, 0)),
      ],
      out_specs=[
          pl.BlockSpec((_BN, H), lambda i: (i, 0)),
          pl.BlockSpec((_BN, H), lambda i: (i, 0)),
      ],
      out_shape=[
          jax.ShapeDtypeStruct((N, H), jnp.float32),
          jax.ShapeDtypeStruct((N, H), jnp.float32),
      ],
  )(x, c0, c1, W0)


def _tc_b_body(s_ref, u_ref, d_ref, b_ref, w_ref, u1_ref):
  h1 = (s_ref[0] + s_ref[1] + u_ref[...]) * d_ref[...] + b_ref[...]
  h1 = jnp.maximum(h1, 0.0)
  u1_ref[...] = jnp.dot(h1, w_ref[...],
                        preferred_element_type=jnp.float32) * d_ref[...]


def _tc_b(s2, u0, d16, b0, W1):
  return pl.pallas_call(
      _tc_b_body,
      grid=(_GRID,),
      in_specs=[
          pl.BlockSpec((NC, _BN, H), lambda i: (0, i, 0)),
          pl.BlockSpec((_BN, H), lambda i: (i, 0)),
          pl.BlockSpec((_BN, H), lambda i: (i, 0)),
          pl.BlockSpec((1, H), lambda i: (0, 0)),
          pl.BlockSpec((H, H), lambda i: (0, 0)),
      ],
      out_specs=pl.BlockSpec((_BN, H), lambda i: (i, 0)),
      out_shape=jax.ShapeDtypeStruct((N, H), jnp.float32),
  )(s2, u0, d16, b0, W1)


def _tc_c_body(s_ref, u_ref, d_ref, b_ref, h_ref):
  h_ref[...] = (s_ref[0] + s_ref[1] + u_ref[...]) * d_ref[...] \
      + b_ref[...]


def _tc_c(s2, u1, d16, b1):
  return pl.pallas_call(
      _tc_c_body,
      grid=(_GRID,),
      in_specs=[
          pl.BlockSpec((NC, _BN, H), lambda i: (0, i, 0)),
          pl.BlockSpec((_BN, H), lambda i: (i, 0)),
          pl.BlockSpec((_BN, H), lambda i: (i, 0)),
          pl.BlockSpec((1, H), lambda i: (0, 0)),
      ],
      out_specs=pl.BlockSpec((_BN, H), lambda i: (i, 0)),
      out_shape=jax.ShapeDtypeStruct((N, H), jnp.float32),
  )(s2, u1, d16, b1)


def _tc_d_body(pa_ref, pb_ref, ga_ref, gb_ref, wl_ref, bl_ref, o_ref):
  cnt = jnp.maximum(ga_ref[...] + gb_ref[...], 1.0)
  p = (pa_ref[...] + pb_ref[...]) / cnt
  o_ref[...] = jax.nn.sigmoid(
      jnp.dot(p, wl_ref[...], preferred_element_type=jnp.float32)
      + bl_ref[...])


def _tc_d(pa, pb, ga, gb, Wl, bl):
  return pl.pallas_call(
      _tc_d_body,
      out_shape=jax.ShapeDtypeStruct((G_PAD, 1), jnp.float32),
  )(pa, pb, ga, gb, Wl, bl)


def kernel(x, edge_index, batch, W0, b0, W1, b1, Wl, bl):
  src = edge_index[0].reshape(E_CH, CH)
  dst = edge_index[1].reshape(E_CH, CH)

  cnt2, gcnt2 = _sc_count(dst, batch)

  u0, d16 = _tc_a(x, cnt2[0].reshape(N_PAD, 1), cnt2[1].reshape(N_PAD, 1),
                  W0)

  s0 = _sc_scatter(u0, src, dst)
  u1 = _tc_b(s0, u0, d16, b0.reshape(1, H), W1)

  s1 = _sc_scatter(u1, src, dst)
  h2 = _tc_c(s1, u1, d16, b1.reshape(1, H))

  p2 = _sc_pool(h2, batch)

  out = _tc_d(p2[0], p2[1],
              gcnt2[0].reshape(G_PAD, 1), gcnt2[1].reshape(G_PAD, 1),
              Wl, bl.reshape(1, 1))
  return out[:G]


# speedup vs baseline: 106.1140x; 1.3678x over previous
import functools

import jax
import jax.numpy as jnp
from jax import lax
from jax.experimental import pallas as pl
from jax.experimental.pallas import tpu as pltpu
from jax.experimental.pallas import tpu_sc as plsc

N = 100000
E = 3200000
H = 16
G = 1000
D_IN = 5

CH = 128
N_CH_FULL = N // CH
N_TAIL = N - N_CH_FULL * CH
N_PAD = (N_CH_FULL + 1) * CH
NP_CH = N_PAD // CH
E_CH = E // CH
G_PAD = 1024
GP_CH = G_PAD // CH

NC = 2
NS = 16
E_CH_SC = E_CH // NC

KG = 8
NG_TOT = E_CH // KG
NG_MAX = (NG_TOT + NC * NS - 1) // (NC * NS)
NG_REM = NG_TOT - (NG_MAX - 1) * NC * NS

_mesh = plsc.VectorSubcoreMesh(core_axis_name="c", subcore_axis_name="s")
_sc_params = pltpu.CompilerParams(use_tc_tiling_on_sc=False)


def _zero_rows(zrows):
  z16 = jnp.zeros((16,), jnp.float32)
  for j in range(CH):
    zrows[j] = z16


@functools.partial(
    pl.kernel,
    out_type=[
        jax.ShapeDtypeStruct((NC, N_PAD), jnp.float32),
        jax.ShapeDtypeStruct((NC, G_PAD), jnp.float32),
    ],
    mesh=_mesh,
    compiler_params=_sc_params,
    scratch_types=[
        pltpu.VMEM_SHARED((N_PAD,), jnp.float32),
        pltpu.VMEM_SHARED((G_PAD,), jnp.float32),
        pltpu.VMEM((2, KG, CH), jnp.int32),
        pltpu.VMEM((CH,), jnp.int32),
        pltpu.VMEM((CH,), jnp.float32),
        pltpu.VMEM((CH,), jnp.float32),
        pltpu.VMEM((N_TAIL,), jnp.int32),
        pltpu.VMEM((N_TAIL,), jnp.float32),
        pltpu.SemaphoreType.DMA,
        pltpu.SemaphoreType.DMA,
        pltpu.SemaphoreType.DMA,
    ],
)
def _sc_count(dst_hbm, batch_hbm, cnt_hbm, gcnt_hbm,
              accd, accg, idbuf, idxb, ones, zb, idx32, ones32,
              semi, sems, semz):
  c = lax.axis_index("c")
  s = lax.axis_index("s")
  w = s * NC + c
  ng = jnp.where(w < NG_REM, NG_MAX, NG_MAX - 1)

  one16 = jnp.full((16,), 1.0, jnp.float32)
  zero16 = jnp.zeros((16,), jnp.float32)
  for j in range(CH // 16):
    ones[pl.ds(j * 16, 16)] = one16
    zb[pl.ds(j * 16, 16)] = zero16
  for j in range(N_TAIL // 16):
    ones32[pl.ds(j * 16, 16)] = one16

  pltpu.make_async_copy(dst_hbm.at[pl.ds(w * KG, KG)], idbuf.at[0],
                        semi).start()

  @pl.loop(0, (NP_CH + NS - 1) // NS)
  def _(i):
    ch = s + i * NS

    @pl.when(ch < NP_CH)
    def _():
      pltpu.make_async_copy(zb, accd.at[pl.ds(ch * CH, CH)], semz).start()

  @pl.when(s < GP_CH)
  def _():
    pltpu.make_async_copy(zb, accg.at[pl.ds(s * CH, CH)], semz).start()

  @pl.loop(0, (NP_CH + NS - 1) // NS)
  def _(i):
    ch = s + i * NS

    @pl.when(ch < NP_CH)
    def _():
      pltpu.make_async_copy(zb, accd.at[pl.ds(ch * CH, CH)], semz).wait()

  @pl.when(s < GP_CH)
  def _():
    pltpu.make_async_copy(zb, accg.at[pl.ds(s * CH, CH)], semz).wait()

  plsc.subcore_barrier()

  @pl.loop(0, NG_MAX)
  def _(i):
    @pl.when(i < ng)
    def _():
      g = w + i * NC * NS
      b = i % 2
      pltpu.make_async_copy(dst_hbm.at[pl.ds(g * KG, KG)], idbuf.at[b],
                            semi).wait()

      @pl.when(i > 0)
      def _():
        for j in range(KG):
          pltpu.make_async_copy(ones, accd.at[idbuf.at[1 - b, j]],
                                sems).wait()

      @pl.when(i + 1 < ng)
      def _():
        g2 = w + (i + 1) * NC * NS
        pltpu.make_async_copy(dst_hbm.at[pl.ds(g2 * KG, KG)],
                              idbuf.at[1 - b], semi).start()

      for j in range(KG):
        pltpu.make_async_copy(ones, accd.at[idbuf.at[b, j]],
                              sems).start(add=True)

  for j in range(KG):
    pltpu.make_async_copy(ones, accd.at[idbuf.at[0, j]], sems).wait()

  @pl.loop(0, (N_CH_FULL + NC * NS - 1) // (NC * NS))
  def _(i):
    ch = w + i * NC * NS

    @pl.when(ch < N_CH_FULL)
    def _():
      pltpu.sync_copy(batch_hbm.at[pl.ds(ch * CH, CH)], idxb)
      pltpu.sync_copy(ones, accg.at[idxb], add=True)

  @pl.when(w == 13)
  def _():
    pltpu.sync_copy(batch_hbm.at[pl.ds(N_CH_FULL * CH, N_TAIL)], idx32)
    pltpu.sync_copy(ones32, accg.at[idx32], add=True)

  plsc.subcore_barrier()

  @pl.loop(0, (NP_CH + NS - 1) // NS)
  def _(i):
    ch = s + i * NS

    @pl.when(ch < NP_CH)
    def _():
      pltpu.make_async_copy(accd.at[pl.ds(ch * CH, CH)],
                            cnt_hbm.at[c, pl.ds(ch * CH, CH)], semz).start()

  @pl.when(s < GP_CH)
  def _():
    pltpu.make_async_copy(accg.at[pl.ds(s * CH, CH)],
                          gcnt_hbm.at[c, pl.ds(s * CH, CH)], semz).start()

  @pl.loop(0, (NP_CH + NS - 1) // NS)
  def _(i):
    ch = s + i * NS

    @pl.when(ch < NP_CH)
    def _():
      pltpu.make_async_copy(accd.at[pl.ds(ch * CH, CH)],
                            cnt_hbm.at[c, pl.ds(ch * CH, CH)], semz).wait()

  @pl.when(s < GP_CH)
  def _():
    pltpu.make_async_copy(accg.at[pl.ds(s * CH, CH)],
                          gcnt_hbm.at[c, pl.ds(s * CH, CH)], semz).wait()


@functools.partial(
    pl.kernel,
    out_type=jax.ShapeDtypeStruct((NC, N_PAD, H), jnp.float32),
    mesh=_mesh,
    compiler_params=_sc_params,
    scratch_types=[
        pltpu.VMEM_SHARED((N_PAD, H), jnp.float32),
        pltpu.VMEM((2, KG, CH), jnp.int32),
        pltpu.VMEM((2, KG, CH), jnp.int32),
        pltpu.VMEM((KG, CH, H), jnp.float32),
        pltpu.VMEM((CH, H), jnp.float32),
        pltpu.SemaphoreType.DMA,
        pltpu.SemaphoreType.DMA,
        pltpu.SemaphoreType.DMA((KG,)),
        pltpu.SemaphoreType.DMA,
        pltpu.SemaphoreType.DMA,
    ],
)
def _sc_scatter(u_hbm, src_hbm, dst_hbm, out_hbm,
                acc, isbuf, idbuf, rows, zrows,
                semis, semid, semg, sems, semz):
  c = lax.axis_index("c")
  s = lax.axis_index("s")
  w = s * NC + c
  ng = jnp.where(w < NG_REM, NG_MAX, NG_MAX - 1)

  _zero_rows(zrows)

  pltpu.make_async_copy(src_hbm.at[pl.ds(w * KG, KG)], isbuf.at[0],
                        semis).start()
  pltpu.make_async_copy(dst_hbm.at[pl.ds(w * KG, KG)], idbuf.at[0],
                        semid).start()

  @pl.loop(0, (NP_CH + NS - 1) // NS)
  def _(i):
    ch = s + i * NS

    @pl.when(ch < NP_CH)
    def _():
      pltpu.make_async_copy(zrows, acc.at[pl.ds(ch * CH, CH)], semz).start()

  @pl.loop(0, (NP_CH + NS - 1) // NS)
  def _(i):
    ch = s + i * NS

    @pl.when(ch < NP_CH)
    def _():
      pltpu.make_async_copy(zrows, acc.at[pl.ds(ch * CH, CH)], semz).wait()

  plsc.subcore_barrier()

  @pl.loop(0, NG_MAX)
  def _(i):
    @pl.when(i < ng)
    def _():
      g = w + i * NC * NS
      b = i % 2
      pltpu.make_async_copy(src_hbm.at[pl.ds(g * KG, KG)], isbuf.at[b],
                            semis).wait()
      pltpu.make_async_copy(dst_hbm.at[pl.ds(g * KG, KG)], idbuf.at[b],
                            semid).wait()

      @pl.when(i > 0)
      def _():
        for j in range(KG):
          pltpu.make_async_copy(rows.at[j], acc.at[idbuf.at[1 - b, j]],
                                sems).wait()

      @pl.when(i + 1 < ng)
      def _():
        g2 = w + (i + 1) * NC * NS
        pltpu.make_async_copy(src_hbm.at[pl.ds(g2 * KG, KG)],
                              isbuf.at[1 - b], semis).start()
        pltpu.make_async_copy(dst_hbm.at[pl.ds(g2 * KG, KG)],
                              idbuf.at[1 - b], semid).start()

      for j in range(KG):
        pltpu.make_async_copy(u_hbm.at[isbuf.at[b, j]], rows.at[j],
                              semg.at[j]).start()
      for j in range(KG):
        pltpu.make_async_copy(u_hbm.at[isbuf.at[b, j]], rows.at[j],
                              semg.at[j]).wait()
        pltpu.make_async_copy(rows.at[j], acc.at[idbuf.at[b, j]],
                              sems).start(add=True)

  for j in range(KG):
    pltpu.make_async_copy(rows.at[j], acc.at[idbuf.at[0, j]], sems).wait()

  plsc.subcore_barrier()

  @pl.loop(0, (NP_CH + NS - 1) // NS)
  def _(i):
    ch = s + i * NS

    @pl.when(ch < NP_CH)
    def _():
      pltpu.make_async_copy(acc.at[pl.ds(ch * CH, CH)],
                            out_hbm.at[c, pl.ds(ch * CH, CH)], semz).start()

  @pl.loop(0, (NP_CH + NS - 1) // NS)
  def _(i):
    ch = s + i * NS

    @pl.when(ch < NP_CH)
    def _():
      pltpu.make_async_copy(acc.at[pl.ds(ch * CH, CH)],
                            out_hbm.at[c, pl.ds(ch * CH, CH)], semz).wait()


@functools.partial(
    pl.kernel,
    out_type=jax.ShapeDtypeStruct((NC, G_PAD, H), jnp.float32),
    mesh=_mesh,
    compiler_params=_sc_params,
    scratch_types=[
        pltpu.VMEM_SHARED((G_PAD, H), jnp.float32),
        pltpu.VMEM((CH,), jnp.int32),
        pltpu.VMEM((CH, H), jnp.float32),
        pltpu.VMEM((CH, H), jnp.float32),
        pltpu.VMEM((N_TAIL,), jnp.int32),
        pltpu.VMEM((N_TAIL, H), jnp.float32),
    ],
)
def _sc_pool(h_hbm, batch_hbm, out_hbm,
             accp, idxb, rows, zrows, idx32, rows32):
  c = lax.axis_index("c")
  s = lax.axis_index("s")
  w = s * NC + c

  _zero_rows(zrows)

  @pl.when(s < GP_CH)
  def _():
    pltpu.sync_copy(zrows, accp.at[pl.ds(s * CH, CH)])

  plsc.subcore_barrier()

  @pl.loop(0, (N_CH_FULL + NC * NS - 1) // (NC * NS))
  def _(i):
    ch = w + i * NC * NS

    @pl.when(ch < N_CH_FULL)
    def _():
      pltpu.sync_copy(batch_hbm.at[pl.ds(ch * CH, CH)], idxb)
      pltpu.sync_copy(h_hbm.at[pl.ds(ch * CH, CH)], rows)
      pltpu.sync_copy(rows, accp.at[idxb], add=True)

  @pl.when(w == 13)
  def _():
    pltpu.sync_copy(batch_hbm.at[pl.ds(N_CH_FULL * CH, N_TAIL)], idx32)
    pltpu.sync_copy(h_hbm.at[pl.ds(N_CH_FULL * CH, N_TAIL)], rows32)
    pltpu.sync_copy(rows32, accp.at[idx32], add=True)

  plsc.subcore_barrier()

  @pl.when(s < GP_CH)
  def _():
    pltpu.sync_copy(accp.at[pl.ds(s * CH, CH)],
                    out_hbm.at[c, pl.ds(s * CH, CH)])


_BN = 8192
_BNV = _BN // 8
_NV = N_PAD // 8
_GRID = (N_PAD + _BN - 1) // _BN


def _tc_a_body(xv_ref, cnt_ref, w_ref, u_ref, d_ref):
  dv = lax.rsqrt(cnt_ref[0] + cnt_ref[1] + 1.0)
  y = jnp.dot(xv_ref[...], w_ref[...], preferred_element_type=jnp.float32)
  u_ref[...] = y * dv
  d_ref[...] = dv


def _tc_a(xv, cv, W0k):
  return pl.pallas_call(
      _tc_a_body,
      grid=(_GRID,),
      in_specs=[
          pl.BlockSpec((_BNV, 8 * D_IN), lambda i: (i, 0)),
          pl.BlockSpec((NC, _BNV, 128), lambda i: (0, i, 0)),
          pl.BlockSpec((8 * D_IN, 128), lambda i: (0, 0)),
      ],
      out_specs=[
          pl.BlockSpec((_BNV, 128), lambda i: (i, 0)),
          pl.BlockSpec((_BNV, 128), lambda i: (i, 0)),
      ],
      out_shape=[
          jax.ShapeDtypeStruct((_NV, 128), jnp.float32),
          jax.ShapeDtypeStruct((_NV, 128), jnp.float32),
      ],
  )(xv, cv, W0k)


def _tc_b_body(s_ref, u_ref, d_ref, b_ref, w_ref, u1_ref):
  h1 = (s_ref[0] + s_ref[1] + u_ref[...]) * d_ref[...] + b_ref[...]
  h1 = jnp.maximum(h1, 0.0)
  u1_ref[...] = jnp.dot(h1, w_ref[...],
                        preferred_element_type=jnp.float32) * d_ref[...]


def _tc_b(s2v, u0v, dv, b0v, W1k):
  return pl.pallas_call(
      _tc_b_body,
      grid=(_GRID,),
      in_specs=[
          pl.BlockSpec((NC, _BNV, 128), lambda i: (0, i, 0)),
          pl.BlockSpec((_BNV, 128), lambda i: (i, 0)),
          pl.BlockSpec((_BNV, 128), lambda i: (i, 0)),
          pl.BlockSpec((1, 128), lambda i: (0, 0)),
          pl.BlockSpec((128, 128), lambda i: (0, 0)),
      ],
      out_specs=pl.BlockSpec((_BNV, 128), lambda i: (i, 0)),
      out_shape=jax.ShapeDtypeStruct((_NV, 128), jnp.float32),
  )(s2v, u0v, dv, b0v, W1k)


def _tc_c_body(s_ref, u_ref, d_ref, b_ref, h_ref):
  h_ref[...] = (s_ref[0] + s_ref[1] + u_ref[...]) * d_ref[...] \
      + b_ref[...]


def _tc_c(s2v, u1v, dv, b1v):
  return pl.pallas_call(
      _tc_c_body,
      grid=(_GRID,),
      in_specs=[
          pl.BlockSpec((NC, _BNV, 128), lambda i: (0, i, 0)),
          pl.BlockSpec((_BNV, 128), lambda i: (i, 0)),
          pl.BlockSpec((_BNV, 128), lambda i: (i, 0)),
          pl.BlockSpec((1, 128), lambda i: (0, 0)),
      ],
      out_specs=pl.BlockSpec((_BNV, 128), lambda i: (i, 0)),
      out_shape=jax.ShapeDtypeStruct((_NV, 128), jnp.float32),
  )(s2v, u1v, dv, b1v)


def _tc_d_body(p_ref, g_ref, wl_ref, bl_ref, o_ref):
  cnt = jnp.maximum(g_ref[0] + g_ref[1], 1.0)
  p = (p_ref[0] + p_ref[1]) / cnt
  o_ref[...] = jax.nn.sigmoid(
      jnp.dot(p, wl_ref[...], preferred_element_type=jnp.float32)
      + bl_ref[...])


def _tc_d(p2v, gv, Wlk, bl):
  return pl.pallas_call(
      _tc_d_body,
      out_shape=jax.ShapeDtypeStruct((G_PAD // 8, 8), jnp.float32),
  )(p2v, gv, Wlk, bl)


def kernel(x, edge_index, batch, W0, b0, W1, b1, Wl, bl):
  src = edge_index[0].reshape(E_CH, CH)
  dst = edge_index[1].reshape(E_CH, CH)

  eye8 = jnp.eye(8, dtype=jnp.float32)
  W0k = jnp.kron(eye8, W0)
  W1k = jnp.kron(eye8, W1)
  Wlk = jnp.kron(eye8, Wl)
  b0v = jnp.tile(b0, 8).reshape(1, 128)
  b1v = jnp.tile(b1, 8).reshape(1, 128)

  xv = jnp.pad(x, ((0, N_PAD - N), (0, 0))).reshape(_NV, 8 * D_IN)

  cnt2, gcnt2 = _sc_count(dst, batch)

  cv = jnp.broadcast_to(cnt2[:, :, None], (NC, N_PAD, H)) \
      .reshape(NC, _NV, 128)
  gv = jnp.broadcast_to(gcnt2[:, :, None], (NC, G_PAD, H)) \
      .reshape(NC, G_PAD // 8, 128)

  u0v, dv = _tc_a(xv, cv, W0k)

  s0 = _sc_scatter(u0v.reshape(N_PAD, H), src, dst)
  u1v = _tc_b(s0.reshape(NC, _NV, 128), u0v, dv, b0v, W1k)

  s1 = _sc_scatter(u1v.reshape(N_PAD, H), src, dst)
  h2v = _tc_c(s1.reshape(NC, _NV, 128), u1v, dv, b1v)

  p2 = _sc_pool(h2v.reshape(N_PAD, H), batch)

  out = _tc_d(p2.reshape(NC, G_PAD // 8, 128), gv, Wlk, bl.reshape(1, 1))
  return out.reshape(G_PAD, 1)[:G]


# speedup vs baseline: 127.1840x; 1.1986x over previous
import functools

import jax
import jax.numpy as jnp
from jax import lax
from jax.experimental import pallas as pl
from jax.experimental.pallas import tpu as pltpu
from jax.experimental.pallas import tpu_sc as plsc

N = 100000
E = 3200000
H = 16
G = 1000
D_IN = 5

CH = 128
N_CH_FULL = N // CH
N_TAIL = N - N_CH_FULL * CH
N_PAD = (N_CH_FULL + 1) * CH
NP_CH = N_PAD // CH
E_CH = E // CH
G_PAD = 1024
GP_CH = G_PAD // CH

NC = 2
NS = 16
E_CH_SC = E_CH // NC

KG = 8
NG_TOT = E_CH // KG
NG_MAX = (NG_TOT + NC * NS - 1) // (NC * NS)
NG_REM = NG_TOT - (NG_MAX - 1) * NC * NS

_mesh = plsc.VectorSubcoreMesh(core_axis_name="c", subcore_axis_name="s")
_sc_params = pltpu.CompilerParams(use_tc_tiling_on_sc=False)


def _zero_rows(zrows):
  z16 = jnp.zeros((16,), jnp.float32)
  for j in range(CH):
    zrows[j] = z16


@functools.partial(
    pl.kernel,
    out_type=[
        jax.ShapeDtypeStruct((NC, N_PAD, H), jnp.float32),
        jax.ShapeDtypeStruct((NC, G_PAD, H), jnp.float32),
    ],
    mesh=_mesh,
    compiler_params=_sc_params,
    scratch_types=[
        pltpu.VMEM_SHARED((N_PAD, H), jnp.float32),
        pltpu.VMEM_SHARED((G_PAD, H), jnp.float32),
        pltpu.VMEM((2, KG, CH), jnp.int32),
        pltpu.VMEM((CH,), jnp.int32),
        pltpu.VMEM((CH, H), jnp.float32),
        pltpu.VMEM((CH, H), jnp.float32),
        pltpu.VMEM((N_TAIL,), jnp.int32),
        pltpu.VMEM((N_TAIL, H), jnp.float32),
        pltpu.SemaphoreType.DMA,
        pltpu.SemaphoreType.DMA,
        pltpu.SemaphoreType.DMA,
    ],
)
def _sc_count(dst_hbm, batch_hbm, cnt_hbm, gcnt_hbm,
              accd, accg, idbuf, idxb, ones, zb, idx32, ones32,
              semi, sems, semz):
  c = lax.axis_index("c")
  s = lax.axis_index("s")
  w = s * NC + c
  ng = jnp.where(w < NG_REM, NG_MAX, NG_MAX - 1)

  one16 = jnp.full((16,), 1.0, jnp.float32)
  zero16 = jnp.zeros((16,), jnp.float32)
  for j in range(CH):
    ones[j] = one16
    zb[j] = zero16
  for j in range(N_TAIL):
    ones32[j] = one16

  pltpu.make_async_copy(dst_hbm.at[pl.ds(w * KG, KG)], idbuf.at[0],
                        semi).start()

  @pl.loop(0, (NP_CH + NS - 1) // NS)
  def _(i):
    ch = s + i * NS

    @pl.when(ch < NP_CH)
    def _():
      pltpu.make_async_copy(zb, accd.at[pl.ds(ch * CH, CH)], semz).start()

  @pl.when(s < GP_CH)
  def _():
    pltpu.make_async_copy(zb, accg.at[pl.ds(s * CH, CH)], semz).start()

  @pl.loop(0, (NP_CH + NS - 1) // NS)
  def _(i):
    ch = s + i * NS

    @pl.when(ch < NP_CH)
    def _():
      pltpu.make_async_copy(zb, accd.at[pl.ds(ch * CH, CH)], semz).wait()

  @pl.when(s < GP_CH)
  def _():
    pltpu.make_async_copy(zb, accg.at[pl.ds(s * CH, CH)], semz).wait()

  plsc.subcore_barrier()

  @pl.loop(0, NG_MAX)
  def _(i):
    @pl.when(i < ng)
    def _():
      g = w + i * NC * NS
      b = i % 2
      pltpu.make_async_copy(dst_hbm.at[pl.ds(g * KG, KG)], idbuf.at[b],
                            semi).wait()

      @pl.when(i > 0)
      def _():
        for j in range(KG):
          pltpu.make_async_copy(ones, accd.at[idbuf.at[1 - b, j]],
                                sems).wait()

      @pl.when(i + 1 < ng)
      def _():
        g2 = w + (i + 1) * NC * NS
        pltpu.make_async_copy(dst_hbm.at[pl.ds(g2 * KG, KG)],
                              idbuf.at[1 - b], semi).start()

      for j in range(KG):
        pltpu.make_async_copy(ones, accd.at[idbuf.at[b, j]],
                              sems).start(add=True)

  for j in range(KG):
    pltpu.make_async_copy(ones, accd.at[idbuf.at[0, j]], sems).wait()

  @pl.loop(0, (N_CH_FULL + NC * NS - 1) // (NC * NS))
  def _(i):
    ch = w + i * NC * NS

    @pl.when(ch < N_CH_FULL)
    def _():
      pltpu.sync_copy(batch_hbm.at[pl.ds(ch * CH, CH)], idxb)
      pltpu.sync_copy(ones, accg.at[idxb], add=True)

  @pl.when(w == 13)
  def _():
    pltpu.sync_copy(batch_hbm.at[pl.ds(N_CH_FULL * CH, N_TAIL)], idx32)
    pltpu.sync_copy(ones32, accg.at[idx32], add=True)

  plsc.subcore_barrier()

  @pl.loop(0, (NP_CH + NS - 1) // NS)
  def _(i):
    ch = s + i * NS

    @pl.when(ch < NP_CH)
    def _():
      pltpu.make_async_copy(accd.at[pl.ds(ch * CH, CH)],
                            cnt_hbm.at[c, pl.ds(ch * CH, CH)], semz).start()

  @pl.when(s < GP_CH)
  def _():
    pltpu.make_async_copy(accg.at[pl.ds(s * CH, CH)],
                          gcnt_hbm.at[c, pl.ds(s * CH, CH)], semz).start()

  @pl.loop(0, (NP_CH + NS - 1) // NS)
  def _(i):
    ch = s + i * NS

    @pl.when(ch < NP_CH)
    def _():
      pltpu.make_async_copy(accd.at[pl.ds(ch * CH, CH)],
                            cnt_hbm.at[c, pl.ds(ch * CH, CH)], semz).wait()

  @pl.when(s < GP_CH)
  def _():
    pltpu.make_async_copy(accg.at[pl.ds(s * CH, CH)],
                          gcnt_hbm.at[c, pl.ds(s * CH, CH)], semz).wait()


@functools.partial(
    pl.kernel,
    out_type=jax.ShapeDtypeStruct((NC, N_PAD, H), jnp.float32),
    mesh=_mesh,
    compiler_params=_sc_params,
    scratch_types=[
        pltpu.VMEM_SHARED((N_PAD, H), jnp.float32),
        pltpu.VMEM((2, KG, CH), jnp.int32),
        pltpu.VMEM((2, KG, CH), jnp.int32),
        pltpu.VMEM((KG, CH, H), jnp.float32),
        pltpu.VMEM((CH, H), jnp.float32),
        pltpu.SemaphoreType.DMA,
        pltpu.SemaphoreType.DMA,
        pltpu.SemaphoreType.DMA((KG,)),
        pltpu.SemaphoreType.DMA,
        pltpu.SemaphoreType.DMA,
    ],
)
def _sc_scatter(u_hbm, src_hbm, dst_hbm, out_hbm,
                acc, isbuf, idbuf, rows, zrows,
                semis, semid, semg, sems, semz):
  c = lax.axis_index("c")
  s = lax.axis_index("s")
  w = s * NC + c
  ng = jnp.where(w < NG_REM, NG_MAX, NG_MAX - 1)

  _zero_rows(zrows)

  pltpu.make_async_copy(src_hbm.at[pl.ds(w * KG, KG)], isbuf.at[0],
                        semis).start()
  pltpu.make_async_copy(dst_hbm.at[pl.ds(w * KG, KG)], idbuf.at[0],
                        semid).start()

  @pl.loop(0, (NP_CH + NS - 1) // NS)
  def _(i):
    ch = s + i * NS

    @pl.when(ch < NP_CH)
    def _():
      pltpu.make_async_copy(zrows, acc.at[pl.ds(ch * CH, CH)], semz).start()

  @pl.loop(0, (NP_CH + NS - 1) // NS)
  def _(i):
    ch = s + i * NS

    @pl.when(ch < NP_CH)
    def _():
      pltpu.make_async_copy(zrows, acc.at[pl.ds(ch * CH, CH)], semz).wait()

  plsc.subcore_barrier()

  @pl.loop(0, NG_MAX)
  def _(i):
    @pl.when(i < ng)
    def _():
      g = w + i * NC * NS
      b = i % 2
      pltpu.make_async_copy(src_hbm.at[pl.ds(g * KG, KG)], isbuf.at[b],
                            semis).wait()
      pltpu.make_async_copy(dst_hbm.at[pl.ds(g * KG, KG)], idbuf.at[b],
                            semid).wait()

      @pl.when(i > 0)
      def _():
        for j in range(KG):
          pltpu.make_async_copy(rows.at[j], acc.at[idbuf.at[1 - b, j]],
                                sems).wait()

      @pl.when(i + 1 < ng)
      def _():
        g2 = w + (i + 1) * NC * NS
        pltpu.make_async_copy(src_hbm.at[pl.ds(g2 * KG, KG)],
                              isbuf.at[1 - b], semis).start()
        pltpu.make_async_copy(dst_hbm.at[pl.ds(g2 * KG, KG)],
                              idbuf.at[1 - b], semid).start()

      for j in range(KG):
        pltpu.make_async_copy(u_hbm.at[isbuf.at[b, j]], rows.at[j],
                              semg.at[j]).start()
      for j in range(KG):
        pltpu.make_async_copy(u_hbm.at[isbuf.at[b, j]], rows.at[j],
                              semg.at[j]).wait()
        pltpu.make_async_copy(rows.at[j], acc.at[idbuf.at[b, j]],
                              sems).start(add=True)

  for j in range(KG):
    pltpu.make_async_copy(rows.at[j], acc.at[idbuf.at[0, j]], sems).wait()

  plsc.subcore_barrier()

  @pl.loop(0, (NP_CH + NS - 1) // NS)
  def _(i):
    ch = s + i * NS

    @pl.when(ch < NP_CH)
    def _():
      pltpu.make_async_copy(acc.at[pl.ds(ch * CH, CH)],
                            out_hbm.at[c, pl.ds(ch * CH, CH)], semz).start()

  @pl.loop(0, (NP_CH + NS - 1) // NS)
  def _(i):
    ch = s + i * NS

    @pl.when(ch < NP_CH)
    def _():
      pltpu.make_async_copy(acc.at[pl.ds(ch * CH, CH)],
                            out_hbm.at[c, pl.ds(ch * CH, CH)], semz).wait()


@functools.partial(
    pl.kernel,
    out_type=jax.ShapeDtypeStruct((NC, G_PAD, H), jnp.float32),
    mesh=_mesh,
    compiler_params=_sc_params,
    scratch_types=[
        pltpu.VMEM_SHARED((G_PAD, H), jnp.float32),
        pltpu.VMEM((CH,), jnp.int32),
        pltpu.VMEM((CH, H), jnp.float32),
        pltpu.VMEM((CH, H), jnp.float32),
        pltpu.VMEM((N_TAIL,), jnp.int32),
        pltpu.VMEM((N_TAIL, H), jnp.float32),
    ],
)
def _sc_pool(h_hbm, batch_hbm, out_hbm,
             accp, idxb, rows, zrows, idx32, rows32):
  c = lax.axis_index("c")
  s = lax.axis_index("s")
  w = s * NC + c

  _zero_rows(zrows)

  @pl.when(s < GP_CH)
  def _():
    pltpu.sync_copy(zrows, accp.at[pl.ds(s * CH, CH)])

  plsc.subcore_barrier()

  @pl.loop(0, (N_CH_FULL + NC * NS - 1) // (NC * NS))
  def _(i):
    ch = w + i * NC * NS

    @pl.when(ch < N_CH_FULL)
    def _():
      pltpu.sync_copy(batch_hbm.at[pl.ds(ch * CH, CH)], idxb)
      pltpu.sync_copy(h_hbm.at[pl.ds(ch * CH, CH)], rows)
      pltpu.sync_copy(rows, accp.at[idxb], add=True)

  @pl.when(w == 13)
  def _():
    pltpu.sync_copy(batch_hbm.at[pl.ds(N_CH_FULL * CH, N_TAIL)], idx32)
    pltpu.sync_copy(h_hbm.at[pl.ds(N_CH_FULL * CH, N_TAIL)], rows32)
    pltpu.sync_copy(rows32, accp.at[idx32], add=True)

  plsc.subcore_barrier()

  @pl.when(s < GP_CH)
  def _():
    pltpu.sync_copy(accp.at[pl.ds(s * CH, CH)],
                    out_hbm.at[c, pl.ds(s * CH, CH)])


_BN = 8192
_BNV = _BN // 8
_NV = N_PAD // 8
_GRID = (N_PAD + _BN - 1) // _BN


def _tc_a_body(xv_ref, cnt_ref, w_ref, u_ref, d_ref):
  dv = lax.rsqrt(cnt_ref[0] + cnt_ref[1] + 1.0)
  y = jnp.dot(xv_ref[...], w_ref[...], preferred_element_type=jnp.float32)
  u_ref[...] = y * dv
  d_ref[...] = dv


def _tc_a(xv, cv, W0k):
  return pl.pallas_call(
      _tc_a_body,
      grid=(_GRID,),
      in_specs=[
          pl.BlockSpec((_BNV, 8 * D_IN), lambda i: (i, 0)),
          pl.BlockSpec((NC, _BNV, 128), lambda i: (0, i, 0)),
          pl.BlockSpec((8 * D_IN, 128), lambda i: (0, 0)),
      ],
      out_specs=[
          pl.BlockSpec((_BNV, 128), lambda i: (i, 0)),
          pl.BlockSpec((_BNV, 128), lambda i: (i, 0)),
      ],
      out_shape=[
          jax.ShapeDtypeStruct((_NV, 128), jnp.float32),
          jax.ShapeDtypeStruct((_NV, 128), jnp.float32),
      ],
  )(xv, cv, W0k)


def _tc_b_body(s_ref, u_ref, d_ref, b_ref, w_ref, u1_ref):
  h1 = (s_ref[0] + s_ref[1] + u_ref[...]) * d_ref[...] + b_ref[...]
  h1 = jnp.maximum(h1, 0.0)
  u1_ref[...] = jnp.dot(h1, w_ref[...],
                        preferred_element_type=jnp.float32) * d_ref[...]


def _tc_b(s2v, u0v, dv, b0v, W1k):
  return pl.pallas_call(
      _tc_b_body,
      grid=(_GRID,),
      in_specs=[
          pl.BlockSpec((NC, _BNV, 128), lambda i: (0, i, 0)),
          pl.BlockSpec((_BNV, 128), lambda i: (i, 0)),
          pl.BlockSpec((_BNV, 128), lambda i: (i, 0)),
          pl.BlockSpec((1, 128), lambda i: (0, 0)),
          pl.BlockSpec((128, 128), lambda i: (0, 0)),
      ],
      out_specs=pl.BlockSpec((_BNV, 128), lambda i: (i, 0)),
      out_shape=jax.ShapeDtypeStruct((_NV, 128), jnp.float32),
  )(s2v, u0v, dv, b0v, W1k)


def _tc_c_body(s_ref, u_ref, d_ref, b_ref, h_ref):
  h_ref[...] = (s_ref[0] + s_ref[1] + u_ref[...]) * d_ref[...] \
      + b_ref[...]


def _tc_c(s2v, u1v, dv, b1v):
  return pl.pallas_call(
      _tc_c_body,
      grid=(_GRID,),
      in_specs=[
          pl.BlockSpec((NC, _BNV, 128), lambda i: (0, i, 0)),
          pl.BlockSpec((_BNV, 128), lambda i: (i, 0)),
          pl.BlockSpec((_BNV, 128), lambda i: (i, 0)),
          pl.BlockSpec((1, 128), lambda i: (0, 0)),
      ],
      out_specs=pl.BlockSpec((_BNV, 128), lambda i: (i, 0)),
      out_shape=jax.ShapeDtypeStruct((_NV, 128), jnp.float32),
  )(s2v, u1v, dv, b1v)


def _tc_d_body(p_ref, g_ref, wl_ref, bl_ref, o_ref):
  cnt = jnp.maximum(g_ref[0] + g_ref[1], 1.0)
  p = (p_ref[0] + p_ref[1]) / cnt
  o_ref[...] = jax.nn.sigmoid(
      jnp.dot(p, wl_ref[...], preferred_element_type=jnp.float32)
      + bl_ref[...])


def _tc_d(p2v, gv, Wlk, bl):
  return pl.pallas_call(
      _tc_d_body,
      out_shape=jax.ShapeDtypeStruct((G_PAD // 8, 8), jnp.float32),
  )(p2v, gv, Wlk, bl)


def kernel(x, edge_index, batch, W0, b0, W1, b1, Wl, bl):
  src = edge_index[0].reshape(E_CH, CH)
  dst = edge_index[1].reshape(E_CH, CH)

  eye8 = jnp.eye(8, dtype=jnp.float32)
  W0k = jnp.kron(eye8, W0)
  W1k = jnp.kron(eye8, W1)
  Wlk = jnp.kron(eye8, Wl)
  b0v = jnp.tile(b0, 8).reshape(1, 128)
  b1v = jnp.tile(b1, 8).reshape(1, 128)

  xv = jnp.pad(x, ((0, N_PAD - N), (0, 0))).reshape(_NV, 8 * D_IN)

  cnt16, gcnt16 = _sc_count(dst, batch)

  cv = cnt16.reshape(NC, _NV, 128)
  gv = gcnt16.reshape(NC, G_PAD // 8, 128)

  u0v, dv = _tc_a(xv, cv, W0k)

  s0 = _sc_scatter(u0v.reshape(N_PAD, H), src, dst)
  u1v = _tc_b(s0.reshape(NC, _NV, 128), u0v, dv, b0v, W1k)

  s1 = _sc_scatter(u1v.reshape(N_PAD, H), src, dst)
  h2v = _tc_c(s1.reshape(NC, _NV, 128), u1v, dv, b1v)

  p2 = _sc_pool(h2v.reshape(N_PAD, H), batch)

  out = _tc_d(p2.reshape(NC, G_PAD // 8, 128), gv, Wlk, bl.reshape(1, 1))
  return out.reshape(G_PAD, 1)[:G]


# speedup vs baseline: 136.3968x; 1.0724x over previous
import functools

import jax
import jax.numpy as jnp
from jax import lax
from jax.experimental import pallas as pl
from jax.experimental.pallas import tpu as pltpu
from jax.experimental.pallas import tpu_sc as plsc

N = 100000
E = 3200000
H = 16
G = 1000
D_IN = 5

CH = 128
N_CH_FULL = N // CH
N_TAIL = N - N_CH_FULL * CH
N_PAD = (N_CH_FULL + 1) * CH
NP_CH = N_PAD // CH
E_CH = E // CH
G_PAD = 1024
GP_CH = G_PAD // CH

NC = 2
NS = 16
E_CH_SC = E_CH // NC

KG = 10
NG_TOT = E_CH // KG
NG_MAX = (NG_TOT + NC * NS - 1) // (NC * NS)
NG_REM = NG_TOT - (NG_MAX - 1) * NC * NS

_mesh = plsc.VectorSubcoreMesh(core_axis_name="c", subcore_axis_name="s")
_sc_params = pltpu.CompilerParams(use_tc_tiling_on_sc=False)


def _zero_rows(zrows):
  z16 = jnp.zeros((16,), jnp.float32)
  for j in range(CH):
    zrows[j] = z16


@functools.partial(
    pl.kernel,
    out_type=[
        jax.ShapeDtypeStruct((NC, N_PAD, H), jnp.float32),
        jax.ShapeDtypeStruct((NC, G_PAD, H), jnp.float32),
    ],
    mesh=_mesh,
    compiler_params=_sc_params,
    scratch_types=[
        pltpu.VMEM_SHARED((N_PAD, H), jnp.float32),
        pltpu.VMEM_SHARED((G_PAD, H), jnp.float32),
        pltpu.VMEM((2, KG, CH), jnp.int32),
        pltpu.VMEM((CH,), jnp.int32),
        pltpu.VMEM((CH, H), jnp.float32),
        pltpu.VMEM((CH, H), jnp.float32),
        pltpu.VMEM((N_TAIL,), jnp.int32),
        pltpu.VMEM((N_TAIL, H), jnp.float32),
        pltpu.SemaphoreType.DMA,
        pltpu.SemaphoreType.DMA,
        pltpu.SemaphoreType.DMA,
    ],
)
def _sc_count(dst_hbm, batch_hbm, cnt_hbm, gcnt_hbm,
              accd, accg, idbuf, idxb, ones, zb, idx32, ones32,
              semi, sems, semz):
  c = lax.axis_index("c")
  s = lax.axis_index("s")
  w = s * NC + c
  ng = jnp.where(w < NG_REM, NG_MAX, NG_MAX - 1)

  one16 = jnp.full((16,), 1.0, jnp.float32)
  zero16 = jnp.zeros((16,), jnp.float32)
  for j in range(CH):
    ones[j] = one16
    zb[j] = zero16
  for j in range(N_TAIL):
    ones32[j] = one16

  pltpu.make_async_copy(dst_hbm.at[pl.ds(w * KG, KG)], idbuf.at[0],
                        semi).start()

  @pl.loop(0, (NP_CH + NS - 1) // NS)
  def _(i):
    ch = s + i * NS

    @pl.when(ch < NP_CH)
    def _():
      pltpu.make_async_copy(zb, accd.at[pl.ds(ch * CH, CH)], semz).start()

  @pl.when(s < GP_CH)
  def _():
    pltpu.make_async_copy(zb, accg.at[pl.ds(s * CH, CH)], semz).start()

  @pl.loop(0, (NP_CH + NS - 1) // NS)
  def _(i):
    ch = s + i * NS

    @pl.when(ch < NP_CH)
    def _():
      pltpu.make_async_copy(zb, accd.at[pl.ds(ch * CH, CH)], semz).wait()

  @pl.when(s < GP_CH)
  def _():
    pltpu.make_async_copy(zb, accg.at[pl.ds(s * CH, CH)], semz).wait()

  plsc.subcore_barrier()

  @pl.loop(0, NG_MAX)
  def _(i):
    @pl.when(i < ng)
    def _():
      g = w + i * NC * NS
      b = i % 2
      pltpu.make_async_copy(dst_hbm.at[pl.ds(g * KG, KG)], idbuf.at[b],
                            semi).wait()

      @pl.when(i > 0)
      def _():
        for j in range(KG):
          pltpu.make_async_copy(ones, accd.at[idbuf.at[1 - b, j]],
                                sems).wait()

      @pl.when(i + 1 < ng)
      def _():
        g2 = w + (i + 1) * NC * NS
        pltpu.make_async_copy(dst_hbm.at[pl.ds(g2 * KG, KG)],
                              idbuf.at[1 - b], semi).start()

      for j in range(KG):
        pltpu.make_async_copy(ones, accd.at[idbuf.at[b, j]],
                              sems).start(add=True)

  for j in range(KG):
    pltpu.make_async_copy(ones, accd.at[idbuf.at[0, j]], sems).wait()

  @pl.loop(0, (N_CH_FULL + NC * NS - 1) // (NC * NS))
  def _(i):
    ch = w + i * NC * NS

    @pl.when(ch < N_CH_FULL)
    def _():
      pltpu.sync_copy(batch_hbm.at[pl.ds(ch * CH, CH)], idxb)
      pltpu.sync_copy(ones, accg.at[idxb], add=True)

  @pl.when(w == 13)
  def _():
    pltpu.sync_copy(batch_hbm.at[pl.ds(N_CH_FULL * CH, N_TAIL)], idx32)
    pltpu.sync_copy(ones32, accg.at[idx32], add=True)

  plsc.subcore_barrier()

  @pl.loop(0, (NP_CH + NS - 1) // NS)
  def _(i):
    ch = s + i * NS

    @pl.when(ch < NP_CH)
    def _():
      pltpu.make_async_copy(accd.at[pl.ds(ch * CH, CH)],
                            cnt_hbm.at[c, pl.ds(ch * CH, CH)], semz).start()

  @pl.when(s < GP_CH)
  def _():
    pltpu.make_async_copy(accg.at[pl.ds(s * CH, CH)],
                          gcnt_hbm.at[c, pl.ds(s * CH, CH)], semz).start()

  @pl.loop(0, (NP_CH + NS - 1) // NS)
  def _(i):
    ch = s + i * NS

    @pl.when(ch < NP_CH)
    def _():
      pltpu.make_async_copy(accd.at[pl.ds(ch * CH, CH)],
                            cnt_hbm.at[c, pl.ds(ch * CH, CH)], semz).wait()

  @pl.when(s < GP_CH)
  def _():
    pltpu.make_async_copy(accg.at[pl.ds(s * CH, CH)],
                          gcnt_hbm.at[c, pl.ds(s * CH, CH)], semz).wait()


@functools.partial(
    pl.kernel,
    out_type=jax.ShapeDtypeStruct((NC, N_PAD, H), jnp.float32),
    mesh=_mesh,
    compiler_params=_sc_params,
    scratch_types=[
        pltpu.VMEM_SHARED((N_PAD, H), jnp.float32),
        pltpu.VMEM((2, KG, CH), jnp.int32),
        pltpu.VMEM((2, KG, CH), jnp.int32),
        pltpu.VMEM((KG, CH, H), jnp.float32),
        pltpu.VMEM((CH, H), jnp.float32),
        pltpu.SemaphoreType.DMA,
        pltpu.SemaphoreType.DMA,
        pltpu.SemaphoreType.DMA((KG,)),
        pltpu.SemaphoreType.DMA,
        pltpu.SemaphoreType.DMA,
    ],
)
def _sc_scatter(u_hbm, src_hbm, dst_hbm, out_hbm,
                acc, isbuf, idbuf, rows, zrows,
                semis, semid, semg, sems, semz):
  c = lax.axis_index("c")
  s = lax.axis_index("s")
  w = s * NC + c
  ng = jnp.where(w < NG_REM, NG_MAX, NG_MAX - 1)

  _zero_rows(zrows)

  pltpu.make_async_copy(src_hbm.at[pl.ds(w * KG, KG)], isbuf.at[0],
                        semis).start()
  pltpu.make_async_copy(dst_hbm.at[pl.ds(w * KG, KG)], idbuf.at[0],
                        semid).start()

  @pl.loop(0, (NP_CH + NS - 1) // NS)
  def _(i):
    ch = s + i * NS

    @pl.when(ch < NP_CH)
    def _():
      pltpu.make_async_copy(zrows, acc.at[pl.ds(ch * CH, CH)], semz).start()

  @pl.loop(0, (NP_CH + NS - 1) // NS)
  def _(i):
    ch = s + i * NS

    @pl.when(ch < NP_CH)
    def _():
      pltpu.make_async_copy(zrows, acc.at[pl.ds(ch * CH, CH)], semz).wait()

  plsc.subcore_barrier()

  @pl.loop(0, NG_MAX)
  def _(i):
    @pl.when(i < ng)
    def _():
      g = w + i * NC * NS
      b = i % 2
      pltpu.make_async_copy(src_hbm.at[pl.ds(g * KG, KG)], isbuf.at[b],
                            semis).wait()
      pltpu.make_async_copy(dst_hbm.at[pl.ds(g * KG, KG)], idbuf.at[b],
                            semid).wait()

      @pl.when(i > 0)
      def _():
        for j in range(KG):
          pltpu.make_async_copy(rows.at[j], acc.at[idbuf.at[1 - b, j]],
                                sems).wait()

      @pl.when(i + 1 < ng)
      def _():
        g2 = w + (i + 1) * NC * NS
        pltpu.make_async_copy(src_hbm.at[pl.ds(g2 * KG, KG)],
                              isbuf.at[1 - b], semis).start()
        pltpu.make_async_copy(dst_hbm.at[pl.ds(g2 * KG, KG)],
                              idbuf.at[1 - b], semid).start()

      for j in range(KG):
        pltpu.make_async_copy(u_hbm.at[isbuf.at[b, j]], rows.at[j],
                              semg.at[j]).start()
      for j in range(KG):
        pltpu.make_async_copy(u_hbm.at[isbuf.at[b, j]], rows.at[j],
                              semg.at[j]).wait()
        pltpu.make_async_copy(rows.at[j], acc.at[idbuf.at[b, j]],
                              sems).start(add=True)

  for j in range(KG):
    pltpu.make_async_copy(rows.at[j], acc.at[idbuf.at[0, j]], sems).wait()

  plsc.subcore_barrier()

  @pl.loop(0, (NP_CH + NS - 1) // NS)
  def _(i):
    ch = s + i * NS

    @pl.when(ch < NP_CH)
    def _():
      pltpu.make_async_copy(acc.at[pl.ds(ch * CH, CH)],
                            out_hbm.at[c, pl.ds(ch * CH, CH)], semz).start()

  @pl.loop(0, (NP_CH + NS - 1) // NS)
  def _(i):
    ch = s + i * NS

    @pl.when(ch < NP_CH)
    def _():
      pltpu.make_async_copy(acc.at[pl.ds(ch * CH, CH)],
                            out_hbm.at[c, pl.ds(ch * CH, CH)], semz).wait()


@functools.partial(
    pl.kernel,
    out_type=jax.ShapeDtypeStruct((NC, G_PAD, H), jnp.float32),
    mesh=_mesh,
    compiler_params=_sc_params,
    scratch_types=[
        pltpu.VMEM_SHARED((G_PAD, H), jnp.float32),
        pltpu.VMEM((CH,), jnp.int32),
        pltpu.VMEM((CH, H), jnp.float32),
        pltpu.VMEM((CH, H), jnp.float32),
        pltpu.VMEM((N_TAIL,), jnp.int32),
        pltpu.VMEM((N_TAIL, H), jnp.float32),
    ],
)
def _sc_pool(h_hbm, batch_hbm, out_hbm,
             accp, idxb, rows, zrows, idx32, rows32):
  c = lax.axis_index("c")
  s = lax.axis_index("s")
  w = s * NC + c

  _zero_rows(zrows)

  @pl.when(s < GP_CH)
  def _():
    pltpu.sync_copy(zrows, accp.at[pl.ds(s * CH, CH)])

  plsc.subcore_barrier()

  @pl.loop(0, (N_CH_FULL + NC * NS - 1) // (NC * NS))
  def _(i):
    ch = w + i * NC * NS

    @pl.when(ch < N_CH_FULL)
    def _():
      pltpu.sync_copy(batch_hbm.at[pl.ds(ch * CH, CH)], idxb)
      pltpu.sync_copy(h_hbm.at[pl.ds(ch * CH, CH)], rows)
      pltpu.sync_copy(rows, accp.at[idxb], add=True)

  @pl.when(w == 13)
  def _():
    pltpu.sync_copy(batch_hbm.at[pl.ds(N_CH_FULL * CH, N_TAIL)], idx32)
    pltpu.sync_copy(h_hbm.at[pl.ds(N_CH_FULL * CH, N_TAIL)], rows32)
    pltpu.sync_copy(rows32, accp.at[idx32], add=True)

  plsc.subcore_barrier()

  @pl.when(s < GP_CH)
  def _():
    pltpu.sync_copy(accp.at[pl.ds(s * CH, CH)],
                    out_hbm.at[c, pl.ds(s * CH, CH)])


_BN = 8192
_BNV = _BN // 8
_NV = N_PAD // 8
_GRID = (N_PAD + _BN - 1) // _BN


def _tc_a_body(xv_ref, cnt_ref, w_ref, u_ref, d_ref):
  dv = lax.rsqrt(cnt_ref[0] + cnt_ref[1] + 1.0)
  y = jnp.dot(xv_ref[...], w_ref[...], preferred_element_type=jnp.float32)
  u_ref[...] = y * dv
  d_ref[...] = dv


def _tc_a(xv, cv, W0k):
  return pl.pallas_call(
      _tc_a_body,
      grid=(_GRID,),
      in_specs=[
          pl.BlockSpec((_BNV, 8 * D_IN), lambda i: (i, 0)),
          pl.BlockSpec((NC, _BNV, 128), lambda i: (0, i, 0)),
          pl.BlockSpec((8 * D_IN, 128), lambda i: (0, 0)),
      ],
      out_specs=[
          pl.BlockSpec((_BNV, 128), lambda i: (i, 0)),
          pl.BlockSpec((_BNV, 128), lambda i: (i, 0)),
      ],
      out_shape=[
          jax.ShapeDtypeStruct((_NV, 128), jnp.float32),
          jax.ShapeDtypeStruct((_NV, 128), jnp.float32),
      ],
  )(xv, cv, W0k)


def _tc_b_body(s_ref, u_ref, d_ref, b_ref, w_ref, u1_ref):
  h1 = (s_ref[0] + s_ref[1] + u_ref[...]) * d_ref[...] + b_ref[...]
  h1 = jnp.maximum(h1, 0.0)
  u1_ref[...] = jnp.dot(h1, w_ref[...],
                        preferred_element_type=jnp.float32) * d_ref[...]


def _tc_b(s2v, u0v, dv, b0v, W1k):
  return pl.pallas_call(
      _tc_b_body,
      grid=(_GRID,),
      in_specs=[
          pl.BlockSpec((NC, _BNV, 128), lambda i: (0, i, 0)),
          pl.BlockSpec((_BNV, 128), lambda i: (i, 0)),
          pl.BlockSpec((_BNV, 128), lambda i: (i, 0)),
          pl.BlockSpec((1, 128), lambda i: (0, 0)),
          pl.BlockSpec((128, 128), lambda i: (0, 0)),
      ],
      out_specs=pl.BlockSpec((_BNV, 128), lambda i: (i, 0)),
      out_shape=jax.ShapeDtypeStruct((_NV, 128), jnp.float32),
  )(s2v, u0v, dv, b0v, W1k)


def _tc_c_body(s_ref, u_ref, d_ref, b_ref, h_ref):
  h_ref[...] = (s_ref[0] + s_ref[1] + u_ref[...]) * d_ref[...] \
      + b_ref[...]


def _tc_c(s2v, u1v, dv, b1v):
  return pl.pallas_call(
      _tc_c_body,
      grid=(_GRID,),
      in_specs=[
          pl.BlockSpec((NC, _BNV, 128), lambda i: (0, i, 0)),
          pl.BlockSpec((_BNV, 128), lambda i: (i, 0)),
          pl.BlockSpec((_BNV, 128), lambda i: (i, 0)),
          pl.BlockSpec((1, 128), lambda i: (0, 0)),
      ],
      out_specs=pl.BlockSpec((_BNV, 128), lambda i: (i, 0)),
      out_shape=jax.ShapeDtypeStruct((_NV, 128), jnp.float32),
  )(s2v, u1v, dv, b1v)


def _tc_d_body(p_ref, g_ref, wl_ref, bl_ref, o_ref):
  cnt = jnp.maximum(g_ref[0] + g_ref[1], 1.0)
  p = (p_ref[0] + p_ref[1]) / cnt
  o_ref[...] = jax.nn.sigmoid(
      jnp.dot(p, wl_ref[...], preferred_element_type=jnp.float32)
      + bl_ref[...])


def _tc_d(p2v, gv, Wlk, bl):
  return pl.pallas_call(
      _tc_d_body,
      out_shape=jax.ShapeDtypeStruct((G_PAD // 8, 8), jnp.float32),
  )(p2v, gv, Wlk, bl)


def kernel(x, edge_index, batch, W0, b0, W1, b1, Wl, bl):
  src = edge_index[0].reshape(E_CH, CH)
  dst = edge_index[1].reshape(E_CH, CH)

  eye8 = jnp.eye(8, dtype=jnp.float32)
  W0k = jnp.kron(eye8, W0)
  W1k = jnp.kron(eye8, W1)
  Wlk = jnp.kron(eye8, Wl)
  b0v = jnp.tile(b0, 8).reshape(1, 128)
  b1v = jnp.tile(b1, 8).reshape(1, 128)

  xv = jnp.pad(x, ((0, N_PAD - N), (0, 0))).reshape(_NV, 8 * D_IN)

  cnt16, gcnt16 = _sc_count(dst, batch)

  cv = cnt16.reshape(NC, _NV, 128)
  gv = gcnt16.reshape(NC, G_PAD // 8, 128)

  u0v, dv = _tc_a(xv, cv, W0k)

  s0 = _sc_scatter(u0v.reshape(N_PAD, H), src, dst)
  u1v = _tc_b(s0.reshape(NC, _NV, 128), u0v, dv, b0v, W1k)

  s1 = _sc_scatter(u1v.reshape(N_PAD, H), src, dst)
  h2v = _tc_c(s1.reshape(NC, _NV, 128), u1v, dv, b1v)

  p2 = _sc_pool(h2v.reshape(N_PAD, H), batch)

  out = _tc_d(p2.reshape(NC, G_PAD // 8, 128), gv, Wlk, bl.reshape(1, 1))
  return out.reshape(G_PAD, 1)[:G]


# speedup vs baseline: 136.5327x; 1.0010x over previous
import functools

import jax
import jax.numpy as jnp
from jax import lax
from jax.experimental import pallas as pl
from jax.experimental.pallas import tpu as pltpu
from jax.experimental.pallas import tpu_sc as plsc

N = 100000
E = 3200000
H = 16
G = 1000
D_IN = 5

CH = 128
N_CH_FULL = N // CH
N_TAIL = N - N_CH_FULL * CH
N_PAD = (N_CH_FULL + 1) * CH
NP_CH = N_PAD // CH
E_CH = E // CH
G_PAD = 1024
GP_CH = G_PAD // CH

NC = 2
NS = 16
E_CH_SC = E_CH // NC

KG = 10
NG_TOT = E_CH // KG
NG_MAX = (NG_TOT + NC * NS - 1) // (NC * NS)
NG_REM = NG_TOT - (NG_MAX - 1) * NC * NS

_mesh = plsc.VectorSubcoreMesh(core_axis_name="c", subcore_axis_name="s")
_sc_params = pltpu.CompilerParams(use_tc_tiling_on_sc=False)


def _zero_rows(zrows):
  z16 = jnp.zeros((16,), jnp.float32)
  for j in range(CH):
    zrows[j] = z16


@functools.partial(
    pl.kernel,
    out_type=[
        jax.ShapeDtypeStruct((NC, N_PAD, H), jnp.float32),
        jax.ShapeDtypeStruct((NC, G_PAD, H), jnp.float32),
    ],
    mesh=_mesh,
    compiler_params=_sc_params,
    scratch_types=[
        pltpu.VMEM_SHARED((N_PAD, H), jnp.float32),
        pltpu.VMEM_SHARED((G_PAD, H), jnp.float32),
        pltpu.VMEM((2, KG, CH), jnp.int32),
        pltpu.VMEM((CH,), jnp.int32),
        pltpu.VMEM((CH, H), jnp.float32),
        pltpu.VMEM((CH, H), jnp.float32),
        pltpu.VMEM((N_TAIL,), jnp.int32),
        pltpu.VMEM((N_TAIL, H), jnp.float32),
        pltpu.SemaphoreType.DMA,
        pltpu.SemaphoreType.DMA,
        pltpu.SemaphoreType.DMA,
    ],
)
def _sc_count(e_hbm, batch_hbm, cnt_hbm, gcnt_hbm,
              accd, accg, idbuf, idxb, ones, zb, idx32, ones32,
              semi, sems, semz):
  c = lax.axis_index("c")
  s = lax.axis_index("s")
  w = s * NC + c
  ng = jnp.where(w < NG_REM, NG_MAX, NG_MAX - 1)

  one16 = jnp.full((16,), 1.0, jnp.float32)
  zero16 = jnp.zeros((16,), jnp.float32)
  for j in range(CH):
    ones[j] = one16
    zb[j] = zero16
  for j in range(N_TAIL):
    ones32[j] = one16

  pltpu.make_async_copy(e_hbm.at[1, pl.ds(w * KG, KG)], idbuf.at[0],
                        semi).start()

  @pl.loop(0, (NP_CH + NS - 1) // NS)
  def _(i):
    ch = s + i * NS

    @pl.when(ch < NP_CH)
    def _():
      pltpu.make_async_copy(zb, accd.at[pl.ds(ch * CH, CH)], semz).start()

  @pl.when(s < GP_CH)
  def _():
    pltpu.make_async_copy(zb, accg.at[pl.ds(s * CH, CH)], semz).start()

  @pl.loop(0, (NP_CH + NS - 1) // NS)
  def _(i):
    ch = s + i * NS

    @pl.when(ch < NP_CH)
    def _():
      pltpu.make_async_copy(zb, accd.at[pl.ds(ch * CH, CH)], semz).wait()

  @pl.when(s < GP_CH)
  def _():
    pltpu.make_async_copy(zb, accg.at[pl.ds(s * CH, CH)], semz).wait()

  plsc.subcore_barrier()

  @pl.loop(0, NG_MAX)
  def _(i):
    @pl.when(i < ng)
    def _():
      g = w + i * NC * NS
      b = i % 2
      pltpu.make_async_copy(e_hbm.at[1, pl.ds(g * KG, KG)], idbuf.at[b],
                            semi).wait()

      @pl.when(i > 0)
      def _():
        for j in range(KG):
          pltpu.make_async_copy(ones, accd.at[idbuf.at[1 - b, j]],
                                sems).wait()

      @pl.when(i + 1 < ng)
      def _():
        g2 = w + (i + 1) * NC * NS
        pltpu.make_async_copy(e_hbm.at[1, pl.ds(g2 * KG, KG)],
                              idbuf.at[1 - b], semi).start()

      for j in range(KG):
        pltpu.make_async_copy(ones, accd.at[idbuf.at[b, j]],
                              sems).start(add=True)

  for j in range(KG):
    pltpu.make_async_copy(ones, accd.at[idbuf.at[0, j]], sems).wait()

  @pl.loop(0, (N_CH_FULL + NC * NS - 1) // (NC * NS))
  def _(i):
    ch = w + i * NC * NS

    @pl.when(ch < N_CH_FULL)
    def _():
      pltpu.sync_copy(batch_hbm.at[pl.ds(ch * CH, CH)], idxb)
      pltpu.sync_copy(ones, accg.at[idxb], add=True)

  @pl.when(w == 13)
  def _():
    pltpu.sync_copy(batch_hbm.at[pl.ds(N_CH_FULL * CH, N_TAIL)], idx32)
    pltpu.sync_copy(ones32, accg.at[idx32], add=True)

  plsc.subcore_barrier()

  @pl.loop(0, (NP_CH + NS - 1) // NS)
  def _(i):
    ch = s + i * NS

    @pl.when(ch < NP_CH)
    def _():
      pltpu.make_async_copy(accd.at[pl.ds(ch * CH, CH)],
                            cnt_hbm.at[c, pl.ds(ch * CH, CH)], semz).start()

  @pl.when(s < GP_CH)
  def _():
    pltpu.make_async_copy(accg.at[pl.ds(s * CH, CH)],
                          gcnt_hbm.at[c, pl.ds(s * CH, CH)], semz).start()

  @pl.loop(0, (NP_CH + NS - 1) // NS)
  def _(i):
    ch = s + i * NS

    @pl.when(ch < NP_CH)
    def _():
      pltpu.make_async_copy(accd.at[pl.ds(ch * CH, CH)],
                            cnt_hbm.at[c, pl.ds(ch * CH, CH)], semz).wait()

  @pl.when(s < GP_CH)
  def _():
    pltpu.make_async_copy(accg.at[pl.ds(s * CH, CH)],
                          gcnt_hbm.at[c, pl.ds(s * CH, CH)], semz).wait()


@functools.partial(
    pl.kernel,
    out_type=jax.ShapeDtypeStruct((NC, N_PAD, H), jnp.float32),
    mesh=_mesh,
    compiler_params=_sc_params,
    scratch_types=[
        pltpu.VMEM_SHARED((N_PAD, H), jnp.float32),
        pltpu.VMEM((2, KG, CH), jnp.int32),
        pltpu.VMEM((2, KG, CH), jnp.int32),
        pltpu.VMEM((KG, CH, H), jnp.float32),
        pltpu.VMEM((CH, H), jnp.float32),
        pltpu.SemaphoreType.DMA,
        pltpu.SemaphoreType.DMA,
        pltpu.SemaphoreType.DMA((KG,)),
        pltpu.SemaphoreType.DMA,
        pltpu.SemaphoreType.DMA,
    ],
)
def _sc_scatter(u_hbm, e_hbm, out_hbm,
                acc, isbuf, idbuf, rows, zrows,
                semis, semid, semg, sems, semz):
  c = lax.axis_index("c")
  s = lax.axis_index("s")
  w = s * NC + c
  ng = jnp.where(w < NG_REM, NG_MAX, NG_MAX - 1)

  _zero_rows(zrows)

  pltpu.make_async_copy(e_hbm.at[0, pl.ds(w * KG, KG)], isbuf.at[0],
                        semis).start()
  pltpu.make_async_copy(e_hbm.at[1, pl.ds(w * KG, KG)], idbuf.at[0],
                        semid).start()

  @pl.loop(0, (NP_CH + NS - 1) // NS)
  def _(i):
    ch = s + i * NS

    @pl.when(ch < NP_CH)
    def _():
      pltpu.make_async_copy(zrows, acc.at[pl.ds(ch * CH, CH)], semz).start()

  @pl.loop(0, (NP_CH + NS - 1) // NS)
  def _(i):
    ch = s + i * NS

    @pl.when(ch < NP_CH)
    def _():
      pltpu.make_async_copy(zrows, acc.at[pl.ds(ch * CH, CH)], semz).wait()

  plsc.subcore_barrier()

  @pl.loop(0, NG_MAX)
  def _(i):
    @pl.when(i < ng)
    def _():
      g = w + i * NC * NS
      b = i % 2
      pltpu.make_async_copy(e_hbm.at[0, pl.ds(g * KG, KG)], isbuf.at[b],
                            semis).wait()
      pltpu.make_async_copy(e_hbm.at[1, pl.ds(g * KG, KG)], idbuf.at[b],
                            semid).wait()

      @pl.when(i > 0)
      def _():
        for j in range(KG):
          pltpu.make_async_copy(rows.at[j], acc.at[idbuf.at[1 - b, j]],
                                sems).wait()

      @pl.when(i + 1 < ng)
      def _():
        g2 = w + (i + 1) * NC * NS
        pltpu.make_async_copy(e_hbm.at[0, pl.ds(g2 * KG, KG)],
                              isbuf.at[1 - b], semis).start()
        pltpu.make_async_copy(e_hbm.at[1, pl.ds(g2 * KG, KG)],
                              idbuf.at[1 - b], semid).start()

      for j in range(KG):
        pltpu.make_async_copy(u_hbm.at[isbuf.at[b, j]], rows.at[j],
                              semg.at[j]).start()
      for j in range(KG):
        pltpu.make_async_copy(u_hbm.at[isbuf.at[b, j]], rows.at[j],
                              semg.at[j]).wait()
        pltpu.make_async_copy(rows.at[j], acc.at[idbuf.at[b, j]],
                              sems).start(add=True)

  for j in range(KG):
    pltpu.make_async_copy(rows.at[j], acc.at[idbuf.at[0, j]], sems).wait()

  plsc.subcore_barrier()

  @pl.loop(0, (NP_CH + NS - 1) // NS)
  def _(i):
    ch = s + i * NS

    @pl.when(ch < NP_CH)
    def _():
      pltpu.make_async_copy(acc.at[pl.ds(ch * CH, CH)],
                            out_hbm.at[c, pl.ds(ch * CH, CH)], semz).start()

  @pl.loop(0, (NP_CH + NS - 1) // NS)
  def _(i):
    ch = s + i * NS

    @pl.when(ch < NP_CH)
    def _():
      pltpu.make_async_copy(acc.at[pl.ds(ch * CH, CH)],
                            out_hbm.at[c, pl.ds(ch * CH, CH)], semz).wait()


@functools.partial(
    pl.kernel,
    out_type=jax.ShapeDtypeStruct((NC, G_PAD, H), jnp.float32),
    mesh=_mesh,
    compiler_params=_sc_params,
    scratch_types=[
        pltpu.VMEM_SHARED((G_PAD, H), jnp.float32),
        pltpu.VMEM((CH,), jnp.int32),
        pltpu.VMEM((CH, H), jnp.float32),
        pltpu.VMEM((CH, H), jnp.float32),
        pltpu.VMEM((N_TAIL,), jnp.int32),
        pltpu.VMEM((N_TAIL, H), jnp.float32),
    ],
)
def _sc_pool(h_hbm, batch_hbm, out_hbm,
             accp, idxb, rows, zrows, idx32, rows32):
  c = lax.axis_index("c")
  s = lax.axis_index("s")
  w = s * NC + c

  _zero_rows(zrows)

  @pl.when(s < GP_CH)
  def _():
    pltpu.sync_copy(zrows, accp.at[pl.ds(s * CH, CH)])

  plsc.subcore_barrier()

  @pl.loop(0, (N_CH_FULL + NC * NS - 1) // (NC * NS))
  def _(i):
    ch = w + i * NC * NS

    @pl.when(ch < N_CH_FULL)
    def _():
      pltpu.sync_copy(batch_hbm.at[pl.ds(ch * CH, CH)], idxb)
      pltpu.sync_copy(h_hbm.at[pl.ds(ch * CH, CH)], rows)
      pltpu.sync_copy(rows, accp.at[idxb], add=True)

  @pl.when(w == 13)
  def _():
    pltpu.sync_copy(batch_hbm.at[pl.ds(N_CH_FULL * CH, N_TAIL)], idx32)
    pltpu.sync_copy(h_hbm.at[pl.ds(N_CH_FULL * CH, N_TAIL)], rows32)
    pltpu.sync_copy(rows32, accp.at[idx32], add=True)

  plsc.subcore_barrier()

  @pl.when(s < GP_CH)
  def _():
    pltpu.sync_copy(accp.at[pl.ds(s * CH, CH)],
                    out_hbm.at[c, pl.ds(s * CH, CH)])


_BN = 8192
_BNV = _BN // 8
_NV = N_PAD // 8
_GRID = (N_PAD + _BN - 1) // _BN


def _tc_a_body(xv_ref, cnt_ref, w_ref, u_ref, d_ref):
  dv = lax.rsqrt(cnt_ref[0] + cnt_ref[1] + 1.0)
  y = jnp.dot(xv_ref[...], w_ref[...], preferred_element_type=jnp.float32)
  u_ref[...] = y * dv
  d_ref[...] = dv


def _tc_a(xv, cv, W0k):
  return pl.pallas_call(
      _tc_a_body,
      grid=(_GRID,),
      in_specs=[
          pl.BlockSpec((_BNV, 8 * D_IN), lambda i: (i, 0)),
          pl.BlockSpec((NC, _BNV, 128), lambda i: (0, i, 0)),
          pl.BlockSpec((8 * D_IN, 128), lambda i: (0, 0)),
      ],
      out_specs=[
          pl.BlockSpec((_BNV, 128), lambda i: (i, 0)),
          pl.BlockSpec((_BNV, 128), lambda i: (i, 0)),
      ],
      out_shape=[
          jax.ShapeDtypeStruct((_NV, 128), jnp.float32),
          jax.ShapeDtypeStruct((_NV, 128), jnp.float32),
      ],
  )(xv, cv, W0k)


def _tc_b_body(s_ref, u_ref, d_ref, b_ref, w_ref, u1_ref):
  h1 = (s_ref[0] + s_ref[1] + u_ref[...]) * d_ref[...] + b_ref[...]
  h1 = jnp.maximum(h1, 0.0)
  u1_ref[...] = jnp.dot(h1, w_ref[...],
                        preferred_element_type=jnp.float32) * d_ref[...]


def _tc_b(s2v, u0v, dv, b0v, W1k):
  return pl.pallas_call(
      _tc_b_body,
      grid=(_GRID,),
      in_specs=[
          pl.BlockSpec((NC, _BNV, 128), lambda i: (0, i, 0)),
          pl.BlockSpec((_BNV, 128), lambda i: (i, 0)),
          pl.BlockSpec((_BNV, 128), lambda i: (i, 0)),
          pl.BlockSpec((1, 128), lambda i: (0, 0)),
          pl.BlockSpec((128, 128), lambda i: (0, 0)),
      ],
      out_specs=pl.BlockSpec((_BNV, 128), lambda i: (i, 0)),
      out_shape=jax.ShapeDtypeStruct((_NV, 128), jnp.float32),
  )(s2v, u0v, dv, b0v, W1k)


def _tc_c_body(s_ref, u_ref, d_ref, b_ref, h_ref):
  h_ref[...] = (s_ref[0] + s_ref[1] + u_ref[...]) * d_ref[...] \
      + b_ref[...]


def _tc_c(s2v, u1v, dv, b1v):
  return pl.pallas_call(
      _tc_c_body,
      grid=(_GRID,),
      in_specs=[
          pl.BlockSpec((NC, _BNV, 128), lambda i: (0, i, 0)),
          pl.BlockSpec((_BNV, 128), lambda i: (i, 0)),
          pl.BlockSpec((_BNV, 128), lambda i: (i, 0)),
          pl.BlockSpec((1, 128), lambda i: (0, 0)),
      ],
      out_specs=pl.BlockSpec((_BNV, 128), lambda i: (i, 0)),
      out_shape=jax.ShapeDtypeStruct((_NV, 128), jnp.float32),
  )(s2v, u1v, dv, b1v)


def _tc_d_body(p_ref, g_ref, wl_ref, bl_ref, o_ref):
  cnt = jnp.maximum(g_ref[0] + g_ref[1], 1.0)
  p = (p_ref[0] + p_ref[1]) / cnt
  o_ref[...] = jax.nn.sigmoid(
      jnp.dot(p, wl_ref[...], preferred_element_type=jnp.float32)
      + bl_ref[...])


def _tc_d(p2v, gv, Wlk, bl):
  return pl.pallas_call(
      _tc_d_body,
      out_shape=jax.ShapeDtypeStruct((G_PAD // 8, 8), jnp.float32),
  )(p2v, gv, Wlk, bl)


def kernel(x, edge_index, batch, W0, b0, W1, b1, Wl, bl):
  e3 = edge_index.reshape(2, E_CH, CH)

  eye8 = jnp.eye(8, dtype=jnp.float32)
  W0k = jnp.kron(eye8, W0)
  W1k = jnp.kron(eye8, W1)
  Wlk = jnp.kron(eye8, Wl)
  b0v = jnp.tile(b0, 8).reshape(1, 128)
  b1v = jnp.tile(b1, 8).reshape(1, 128)

  xv = jnp.pad(x, ((0, N_PAD - N), (0, 0))).reshape(_NV, 8 * D_IN)

  cnt16, gcnt16 = _sc_count(e3, batch)

  cv = cnt16.reshape(NC, _NV, 128)
  gv = gcnt16.reshape(NC, G_PAD // 8, 128)

  u0v, dv = _tc_a(xv, cv, W0k)

  s0 = _sc_scatter(u0v.reshape(N_PAD, H), e3)
  u1v = _tc_b(s0.reshape(NC, _NV, 128), u0v, dv, b0v, W1k)

  s1 = _sc_scatter(u1v.reshape(N_PAD, H), e3)
  h2v = _tc_c(s1.reshape(NC, _NV, 128), u1v, dv, b1v)

  p2 = _sc_pool(h2v.reshape(N_PAD, H), batch)

  out = _tc_d(p2.reshape(NC, G_PAD // 8, 128), gv, Wlk, bl.reshape(1, 1))
  return out.reshape(G_PAD, 1)[:G]


# speedup vs baseline: 139.7306x; 1.0234x over previous
import functools

import jax
import jax.numpy as jnp
from jax import lax
from jax.experimental import pallas as pl
from jax.experimental.pallas import tpu as pltpu
from jax.experimental.pallas import tpu_sc as plsc

N = 100000
E = 3200000
H = 16
G = 1000
D_IN = 5

CH = 128
N_CH_FULL = N // CH
N_TAIL = N - N_CH_FULL * CH
N_PAD = (N_CH_FULL + 1) * CH
NP_CH = N_PAD // CH
E_CH = E // CH
G_PAD = 1024
GP_CH = G_PAD // CH

NC = 2
NS = 16
E_CH_SC = E_CH // NC

KG = 10
NG_TOT = E_CH // KG
NG_MAX = (NG_TOT + NC * NS - 1) // (NC * NS)
NG_REM = NG_TOT - (NG_MAX - 1) * NC * NS

_mesh = plsc.VectorSubcoreMesh(core_axis_name="c", subcore_axis_name="s")
_sc_params = pltpu.CompilerParams(use_tc_tiling_on_sc=False)


def _zero_rows(zrows):
  z16 = jnp.zeros((16,), jnp.float32)
  for j in range(CH):
    zrows[j] = z16


@functools.partial(
    pl.kernel,
    out_type=[
        jax.ShapeDtypeStruct((NC, N_PAD, H), jnp.float32),
        jax.ShapeDtypeStruct((NC, G_PAD, H), jnp.float32),
    ],
    mesh=_mesh,
    compiler_params=_sc_params,
    scratch_types=[
        pltpu.VMEM_SHARED((N_PAD, H), jnp.float32),
        pltpu.VMEM_SHARED((G_PAD, H), jnp.float32),
        pltpu.VMEM((2, KG, CH), jnp.int32),
        pltpu.VMEM((CH,), jnp.int32),
        pltpu.VMEM((CH, H), jnp.float32),
        pltpu.VMEM((CH, H), jnp.float32),
        pltpu.VMEM((N_TAIL,), jnp.int32),
        pltpu.VMEM((N_TAIL, H), jnp.float32),
        pltpu.SemaphoreType.DMA,
        pltpu.SemaphoreType.DMA,
        pltpu.SemaphoreType.DMA,
    ],
)
def _sc_count(e_hbm, batch_hbm, cnt_hbm, gcnt_hbm,
              accd, accg, idbuf, idxb, ones, zb, idx32, ones32,
              semi, sems, semz):
  c = lax.axis_index("c")
  s = lax.axis_index("s")
  w = s * NC + c
  ng = jnp.where(w < NG_REM, NG_MAX, NG_MAX - 1)

  one16 = jnp.full((16,), 1.0, jnp.float32)
  zero16 = jnp.zeros((16,), jnp.float32)
  for j in range(CH):
    ones[j] = one16
    zb[j] = zero16
  for j in range(N_TAIL):
    ones32[j] = one16

  pltpu.make_async_copy(e_hbm.at[1, pl.ds(w * KG, KG)], idbuf.at[0],
                        semi).start()

  @pl.loop(0, (NP_CH + NS - 1) // NS)
  def _(i):
    ch = s + i * NS

    @pl.when(ch < NP_CH)
    def _():
      pltpu.make_async_copy(zb, accd.at[pl.ds(ch * CH, CH)], semz).start()

  @pl.when(s < GP_CH)
  def _():
    pltpu.make_async_copy(zb, accg.at[pl.ds(s * CH, CH)], semz).start()

  @pl.loop(0, (NP_CH + NS - 1) // NS)
  def _(i):
    ch = s + i * NS

    @pl.when(ch < NP_CH)
    def _():
      pltpu.make_async_copy(zb, accd.at[pl.ds(ch * CH, CH)], semz).wait()

  @pl.when(s < GP_CH)
  def _():
    pltpu.make_async_copy(zb, accg.at[pl.ds(s * CH, CH)], semz).wait()

  plsc.subcore_barrier()

  @pl.loop(0, NG_MAX)
  def _(i):
    @pl.when(i < ng)
    def _():
      g = w + i * NC * NS
      b = i % 2
      pltpu.make_async_copy(e_hbm.at[1, pl.ds(g * KG, KG)], idbuf.at[b],
                            semi).wait()

      @pl.when(i > 0)
      def _():
        for j in range(KG):
          pltpu.make_async_copy(ones, accd.at[idbuf.at[1 - b, j]],
                                sems).wait()

      @pl.when(i + 1 < ng)
      def _():
        g2 = w + (i + 1) * NC * NS
        pltpu.make_async_copy(e_hbm.at[1, pl.ds(g2 * KG, KG)],
                              idbuf.at[1 - b], semi).start()

      for j in range(KG):
        pltpu.make_async_copy(ones, accd.at[idbuf.at[b, j]],
                              sems).start(add=True)

  for j in range(KG):
    pltpu.make_async_copy(ones, accd.at[idbuf.at[0, j]], sems).wait()

  @pl.loop(0, (N_CH_FULL + NC * NS - 1) // (NC * NS))
  def _(i):
    ch = w + i * NC * NS

    @pl.when(ch < N_CH_FULL)
    def _():
      pltpu.sync_copy(batch_hbm.at[pl.ds(ch * CH, CH)], idxb)
      pltpu.sync_copy(ones, accg.at[idxb], add=True)

  @pl.when(w == 13)
  def _():
    pltpu.sync_copy(batch_hbm.at[pl.ds(N_CH_FULL * CH, N_TAIL)], idx32)
    pltpu.sync_copy(ones32, accg.at[idx32], add=True)

  plsc.subcore_barrier()

  @pl.loop(0, (NP_CH + NS - 1) // NS)
  def _(i):
    ch = s + i * NS

    @pl.when(ch < NP_CH)
    def _():
      pltpu.make_async_copy(accd.at[pl.ds(ch * CH, CH)],
                            cnt_hbm.at[c, pl.ds(ch * CH, CH)], semz).start()

  @pl.when(s < GP_CH)
  def _():
    pltpu.make_async_copy(accg.at[pl.ds(s * CH, CH)],
                          gcnt_hbm.at[c, pl.ds(s * CH, CH)], semz).start()

  @pl.loop(0, (NP_CH + NS - 1) // NS)
  def _(i):
    ch = s + i * NS

    @pl.when(ch < NP_CH)
    def _():
      pltpu.make_async_copy(accd.at[pl.ds(ch * CH, CH)],
                            cnt_hbm.at[c, pl.ds(ch * CH, CH)], semz).wait()

  @pl.when(s < GP_CH)
  def _():
    pltpu.make_async_copy(accg.at[pl.ds(s * CH, CH)],
                          gcnt_hbm.at[c, pl.ds(s * CH, CH)], semz).wait()


@functools.partial(
    pl.kernel,
    out_type=jax.ShapeDtypeStruct((NC, N_PAD, H), jnp.float32),
    mesh=_mesh,
    compiler_params=_sc_params,
    scratch_types=[
        pltpu.VMEM_SHARED((N_PAD, H), jnp.float32),
        pltpu.VMEM((2, KG, CH), jnp.int32),
        pltpu.VMEM((2, KG, CH), jnp.int32),
        pltpu.VMEM((KG, CH, H), jnp.float32),
        pltpu.VMEM((CH, H), jnp.float32),
        pltpu.SemaphoreType.DMA,
        pltpu.SemaphoreType.DMA,
        pltpu.SemaphoreType.DMA((KG,)),
        pltpu.SemaphoreType.DMA,
        pltpu.SemaphoreType.DMA,
    ],
)
def _sc_scatter(u_hbm, e_hbm, out_hbm,
                acc, isbuf, idbuf, rows, zrows,
                semis, semid, semg, sems, semz):
  c = lax.axis_index("c")
  s = lax.axis_index("s")
  w = s * NC + c
  ng = jnp.where(w < NG_REM, NG_MAX, NG_MAX - 1)

  _zero_rows(zrows)

  pltpu.make_async_copy(e_hbm.at[0, pl.ds(w * KG, KG)], isbuf.at[0],
                        semis).start()
  pltpu.make_async_copy(e_hbm.at[1, pl.ds(w * KG, KG)], idbuf.at[0],
                        semid).start()

  @pl.loop(0, (NP_CH + NS - 1) // NS)
  def _(i):
    ch = s + i * NS

    @pl.when(ch < NP_CH)
    def _():
      pltpu.make_async_copy(zrows, acc.at[pl.ds(ch * CH, CH)], semz).start()

  @pl.loop(0, (NP_CH + NS - 1) // NS)
  def _(i):
    ch = s + i * NS

    @pl.when(ch < NP_CH)
    def _():
      pltpu.make_async_copy(zrows, acc.at[pl.ds(ch * CH, CH)], semz).wait()

  plsc.subcore_barrier()

  @pl.loop(0, NG_MAX)
  def _(i):
    @pl.when(i < ng)
    def _():
      g = w + i * NC * NS
      b = i % 2
      pltpu.make_async_copy(e_hbm.at[0, pl.ds(g * KG, KG)], isbuf.at[b],
                            semis).wait()
      pltpu.make_async_copy(e_hbm.at[1, pl.ds(g * KG, KG)], idbuf.at[b],
                            semid).wait()

      @pl.when(i > 0)
      def _():
        for j in range(KG):
          pltpu.make_async_copy(rows.at[j], acc.at[idbuf.at[1 - b, j]],
                                sems).wait()

      @pl.when(i + 1 < ng)
      def _():
        g2 = w + (i + 1) * NC * NS
        pltpu.make_async_copy(e_hbm.at[0, pl.ds(g2 * KG, KG)],
                              isbuf.at[1 - b], semis).start()
        pltpu.make_async_copy(e_hbm.at[1, pl.ds(g2 * KG, KG)],
                              idbuf.at[1 - b], semid).start()

      for j in range(KG):
        pltpu.make_async_copy(u_hbm.at[isbuf.at[b, j]], rows.at[j],
                              semg.at[j]).start()
      for j in range(KG):
        pltpu.make_async_copy(u_hbm.at[isbuf.at[b, j]], rows.at[j],
                              semg.at[j]).wait()
        pltpu.make_async_copy(rows.at[j], acc.at[idbuf.at[b, j]],
                              sems).start(add=True)

  for j in range(KG):
    pltpu.make_async_copy(rows.at[j], acc.at[idbuf.at[0, j]], sems).wait()

  plsc.subcore_barrier()

  @pl.loop(0, (NP_CH + NS - 1) // NS)
  def _(i):
    ch = s + i * NS

    @pl.when(ch < NP_CH)
    def _():
      pltpu.make_async_copy(acc.at[pl.ds(ch * CH, CH)],
                            out_hbm.at[c, pl.ds(ch * CH, CH)], semz).start()

  @pl.loop(0, (NP_CH + NS - 1) // NS)
  def _(i):
    ch = s + i * NS

    @pl.when(ch < NP_CH)
    def _():
      pltpu.make_async_copy(acc.at[pl.ds(ch * CH, CH)],
                            out_hbm.at[c, pl.ds(ch * CH, CH)], semz).wait()


@functools.partial(
    pl.kernel,
    out_type=jax.ShapeDtypeStruct((NC, G_PAD, H), jnp.float32),
    mesh=_mesh,
    compiler_params=_sc_params,
    scratch_types=[
        pltpu.VMEM_SHARED((G_PAD, H), jnp.float32),
        pltpu.VMEM((2, CH), jnp.int32),
        pltpu.VMEM((2, CH, H), jnp.float32),
        pltpu.VMEM((CH, H), jnp.float32),
        pltpu.VMEM((N_TAIL,), jnp.int32),
        pltpu.VMEM((N_TAIL, H), jnp.float32),
        pltpu.SemaphoreType.DMA,
        pltpu.SemaphoreType.DMA,
        pltpu.SemaphoreType.DMA,
    ],
)
def _sc_pool(h_hbm, batch_hbm, out_hbm,
             accp, idxb, rows, zrows, idx32, rows32, semi, semr, sems):
  c = lax.axis_index("c")
  s = lax.axis_index("s")
  w = s * NC + c

  _zero_rows(zrows)

  pltpu.make_async_copy(batch_hbm.at[pl.ds(w * CH, CH)], idxb.at[0],
                        semi).start()
  pltpu.make_async_copy(h_hbm.at[pl.ds(w * CH, CH)], rows.at[0],
                        semr).start()

  @pl.when(s < GP_CH)
  def _():
    pltpu.sync_copy(zrows, accp.at[pl.ds(s * CH, CH)])

  plsc.subcore_barrier()

  @pl.loop(0, (N_CH_FULL + NC * NS - 1) // (NC * NS))
  def _(i):
    ch = w + i * NC * NS

    @pl.when(ch < N_CH_FULL)
    def _():
      b = i % 2
      pltpu.make_async_copy(batch_hbm.at[pl.ds(ch * CH, CH)], idxb.at[b],
                            semi).wait()
      pltpu.make_async_copy(h_hbm.at[pl.ds(ch * CH, CH)], rows.at[b],
                            semr).wait()

      @pl.when(i > 0)
      def _():
        pltpu.make_async_copy(rows.at[1 - b], accp.at[idxb.at[1 - b]],
                              sems).wait()

      ch2 = ch + NC * NS

      @pl.when(ch2 < N_CH_FULL)
      def _():
        pltpu.make_async_copy(batch_hbm.at[pl.ds(ch2 * CH, CH)],
                              idxb.at[1 - b], semi).start()
        pltpu.make_async_copy(h_hbm.at[pl.ds(ch2 * CH, CH)],
                              rows.at[1 - b], semr).start()

      pltpu.make_async_copy(rows.at[b], accp.at[idxb.at[b]],
                            sems).start(add=True)

  pltpu.make_async_copy(rows.at[0], accp.at[idxb.at[0]], sems).wait()

  @pl.when(w == 13)
  def _():
    pltpu.sync_copy(batch_hbm.at[pl.ds(N_CH_FULL * CH, N_TAIL)], idx32)
    pltpu.sync_copy(h_hbm.at[pl.ds(N_CH_FULL * CH, N_TAIL)], rows32)
    pltpu.sync_copy(rows32, accp.at[idx32], add=True)

  plsc.subcore_barrier()

  @pl.when(s < GP_CH)
  def _():
    pltpu.sync_copy(accp.at[pl.ds(s * CH, CH)],
                    out_hbm.at[c, pl.ds(s * CH, CH)])


_BN = 8192
_BNV = _BN // 8
_NV = N_PAD // 8
_GRID = (N_PAD + _BN - 1) // _BN


def _tc_a_body(xv_ref, cnt_ref, w_ref, u_ref, d_ref):
  dv = lax.rsqrt(cnt_ref[0] + cnt_ref[1] + 1.0)
  y = jnp.dot(xv_ref[...], w_ref[...], preferred_element_type=jnp.float32)
  u_ref[...] = y * dv
  d_ref[...] = dv


def _tc_a(xv, cv, W0k):
  return pl.pallas_call(
      _tc_a_body,
      grid=(_GRID,),
      in_specs=[
          pl.BlockSpec((_BNV, 8 * D_IN), lambda i: (i, 0)),
          pl.BlockSpec((NC, _BNV, 128), lambda i: (0, i, 0)),
          pl.BlockSpec((8 * D_IN, 128), lambda i: (0, 0)),
      ],
      out_specs=[
          pl.BlockSpec((_BNV, 128), lambda i: (i, 0)),
          pl.BlockSpec((_BNV, 128), lambda i: (i, 0)),
      ],
      out_shape=[
          jax.ShapeDtypeStruct((_NV, 128), jnp.float32),
          jax.ShapeDtypeStruct((_NV, 128), jnp.float32),
      ],
  )(xv, cv, W0k)


def _tc_b_body(s_ref, u_ref, d_ref, b_ref, w_ref, u1_ref):
  h1 = (s_ref[0] + s_ref[1] + u_ref[...]) * d_ref[...] + b_ref[...]
  h1 = jnp.maximum(h1, 0.0)
  u1_ref[...] = jnp.dot(h1, w_ref[...],
                        preferred_element_type=jnp.float32) * d_ref[...]


def _tc_b(s2v, u0v, dv, b0v, W1k):
  return pl.pallas_call(
      _tc_b_body,
      grid=(_GRID,),
      in_specs=[
          pl.BlockSpec((NC, _BNV, 128), lambda i: (0, i, 0)),
          pl.BlockSpec((_BNV, 128), lambda i: (i, 0)),
          pl.BlockSpec((_BNV, 128), lambda i: (i, 0)),
          pl.BlockSpec((1, 128), lambda i: (0, 0)),
          pl.BlockSpec((128, 128), lambda i: (0, 0)),
      ],
      out_specs=pl.BlockSpec((_BNV, 128), lambda i: (i, 0)),
      out_shape=jax.ShapeDtypeStruct((_NV, 128), jnp.float32),
  )(s2v, u0v, dv, b0v, W1k)


def _tc_c_body(s_ref, u_ref, d_ref, b_ref, h_ref):
  h_ref[...] = (s_ref[0] + s_ref[1] + u_ref[...]) * d_ref[...] \
      + b_ref[...]


def _tc_c(s2v, u1v, dv, b1v):
  return pl.pallas_call(
      _tc_c_body,
      grid=(_GRID,),
      in_specs=[
          pl.BlockSpec((NC, _BNV, 128), lambda i: (0, i, 0)),
          pl.BlockSpec((_BNV, 128), lambda i: (i, 0)),
          pl.BlockSpec((_BNV, 128), lambda i: (i, 0)),
          pl.BlockSpec((1, 128), lambda i: (0, 0)),
      ],
      out_specs=pl.BlockSpec((_BNV, 128), lambda i: (i, 0)),
      out_shape=jax.ShapeDtypeStruct((_NV, 128), jnp.float32),
  )(s2v, u1v, dv, b1v)


def _tc_d_body(p_ref, g_ref, wl_ref, bl_ref, o_ref):
  cnt = jnp.maximum(g_ref[0] + g_ref[1], 1.0)
  p = (p_ref[0] + p_ref[1]) / cnt
  o_ref[...] = jax.nn.sigmoid(
      jnp.dot(p, wl_ref[...], preferred_element_type=jnp.float32)
      + bl_ref[...])


def _tc_d(p2v, gv, Wlk, bl):
  return pl.pallas_call(
      _tc_d_body,
      out_shape=jax.ShapeDtypeStruct((G_PAD // 8, 8), jnp.float32),
  )(p2v, gv, Wlk, bl)


def kernel(x, edge_index, batch, W0, b0, W1, b1, Wl, bl):
  e3 = edge_index.reshape(2, E_CH, CH)

  eye8 = jnp.eye(8, dtype=jnp.float32)
  W0k = jnp.kron(eye8, W0)
  W1k = jnp.kron(eye8, W1)
  Wlk = jnp.kron(eye8, Wl)
  b0v = jnp.tile(b0, 8).reshape(1, 128)
  b1v = jnp.tile(b1, 8).reshape(1, 128)

  xv = jnp.pad(x, ((0, N_PAD - N), (0, 0))).reshape(_NV, 8 * D_IN)

  cnt16, gcnt16 = _sc_count(e3, batch)

  cv = cnt16.reshape(NC, _NV, 128)
  gv = gcnt16.reshape(NC, G_PAD // 8, 128)

  u0v, dv = _tc_a(xv, cv, W0k)

  s0 = _sc_scatter(u0v.reshape(N_PAD, H), e3)
  u1v = _tc_b(s0.reshape(NC, _NV, 128), u0v, dv, b0v, W1k)

  s1 = _sc_scatter(u1v.reshape(N_PAD, H), e3)
  h2v = _tc_c(s1.reshape(NC, _NV, 128), u1v, dv, b1v)

  p2 = _sc_pool(h2v.reshape(N_PAD, H), batch)

  out = _tc_d(p2.reshape(NC, G_PAD // 8, 128), gv, Wlk, bl.reshape(1, 1))
  return out.reshape(G_PAD, 1)[:G]


# speedup vs baseline: 140.7537x; 1.0073x over previous
import functools

import jax
import jax.numpy as jnp
from jax import lax
from jax.experimental import pallas as pl
from jax.experimental.pallas import tpu as pltpu
from jax.experimental.pallas import tpu_sc as plsc

N = 100000
E = 3200000
H = 16
G = 1000
D_IN = 5

CH = 128
N_CH_FULL = N // CH
N_TAIL = N - N_CH_FULL * CH
N_PAD = (N_CH_FULL + 1) * CH
NP_CH = N_PAD // CH
E_CH = E // CH
G_PAD = 1024
GP_CH = G_PAD // CH

NC = 2
NS = 16
E_CH_SC = E_CH // NC

KG = 10
NG_TOT = E_CH // KG
NG_MAX = (NG_TOT + NC * NS - 1) // (NC * NS)
NG_REM = NG_TOT - (NG_MAX - 1) * NC * NS

_mesh = plsc.VectorSubcoreMesh(core_axis_name="c", subcore_axis_name="s")
_sc_params = pltpu.CompilerParams(use_tc_tiling_on_sc=False)


def _zero_rows(zrows):
  z16 = jnp.zeros((16,), jnp.float32)
  for j in range(CH):
    zrows[j] = z16


@functools.partial(
    pl.kernel,
    out_type=[
        jax.ShapeDtypeStruct((NC, N_PAD, H), jnp.float32),
        jax.ShapeDtypeStruct((NC, G_PAD, H), jnp.float32),
    ],
    mesh=_mesh,
    compiler_params=_sc_params,
    scratch_types=[
        pltpu.VMEM_SHARED((N_PAD, H), jnp.float32),
        pltpu.VMEM_SHARED((G_PAD, H), jnp.float32),
        pltpu.VMEM((2, KG, CH), jnp.int32),
        pltpu.VMEM((2, CH), jnp.int32),
        pltpu.VMEM((CH, H), jnp.float32),
        pltpu.VMEM((CH, H), jnp.float32),
        pltpu.VMEM((N_TAIL,), jnp.int32),
        pltpu.VMEM((N_TAIL, H), jnp.float32),
        pltpu.SemaphoreType.DMA,
        pltpu.SemaphoreType.DMA,
        pltpu.SemaphoreType.DMA,
        pltpu.SemaphoreType.DMA,
    ],
)
def _sc_count(e_hbm, batch_hbm, cnt_hbm, gcnt_hbm,
              accd, accg, idbuf, idxb, ones, zb, idx32, ones32,
              semi, sems, semz, semb):
  c = lax.axis_index("c")
  s = lax.axis_index("s")
  w = s * NC + c
  ng = jnp.where(w < NG_REM, NG_MAX, NG_MAX - 1)

  one16 = jnp.full((16,), 1.0, jnp.float32)
  zero16 = jnp.zeros((16,), jnp.float32)
  for j in range(CH):
    ones[j] = one16
    zb[j] = zero16
  for j in range(N_TAIL):
    ones32[j] = one16

  pltpu.make_async_copy(e_hbm.at[1, pl.ds(w * KG, KG)], idbuf.at[0],
                        semi).start()
  pltpu.make_async_copy(batch_hbm.at[pl.ds(w * CH, CH)], idxb.at[0],
                        semb).start()

  @pl.loop(0, (NP_CH + NS - 1) // NS)
  def _(i):
    ch = s + i * NS

    @pl.when(ch < NP_CH)
    def _():
      pltpu.make_async_copy(zb, accd.at[pl.ds(ch * CH, CH)], semz).start()

  @pl.when(s < GP_CH)
  def _():
    pltpu.make_async_copy(zb, accg.at[pl.ds(s * CH, CH)], semz).start()

  @pl.loop(0, (NP_CH + NS - 1) // NS)
  def _(i):
    ch = s + i * NS

    @pl.when(ch < NP_CH)
    def _():
      pltpu.make_async_copy(zb, accd.at[pl.ds(ch * CH, CH)], semz).wait()

  @pl.when(s < GP_CH)
  def _():
    pltpu.make_async_copy(zb, accg.at[pl.ds(s * CH, CH)], semz).wait()

  plsc.subcore_barrier()

  @pl.loop(0, NG_MAX)
  def _(i):
    @pl.when(i < ng)
    def _():
      g = w + i * NC * NS
      b = i % 2
      pltpu.make_async_copy(e_hbm.at[1, pl.ds(g * KG, KG)], idbuf.at[b],
                            semi).wait()

      @pl.when(i > 0)
      def _():
        for j in range(KG):
          pltpu.make_async_copy(ones, accd.at[idbuf.at[1 - b, j]],
                                sems).wait()

      @pl.when(i + 1 < ng)
      def _():
        g2 = w + (i + 1) * NC * NS
        pltpu.make_async_copy(e_hbm.at[1, pl.ds(g2 * KG, KG)],
                              idbuf.at[1 - b], semi).start()

      for j in range(KG):
        pltpu.make_async_copy(ones, accd.at[idbuf.at[b, j]],
                              sems).start(add=True)

  for j in range(KG):
    pltpu.make_async_copy(ones, accd.at[idbuf.at[0, j]], sems).wait()

  @pl.loop(0, (N_CH_FULL + NC * NS - 1) // (NC * NS))
  def _(i):
    ch = w + i * NC * NS

    @pl.when(ch < N_CH_FULL)
    def _():
      b = i % 2
      pltpu.make_async_copy(batch_hbm.at[pl.ds(ch * CH, CH)], idxb.at[b],
                            semb).wait()

      @pl.when(i > 0)
      def _():
        pltpu.make_async_copy(ones, accg.at[idxb.at[1 - b]], sems).wait()

      ch2 = ch + NC * NS

      @pl.when(ch2 < N_CH_FULL)
      def _():
        pltpu.make_async_copy(batch_hbm.at[pl.ds(ch2 * CH, CH)],
                              idxb.at[1 - b], semb).start()

      pltpu.make_async_copy(ones, accg.at[idxb.at[b]], sems).start(add=True)

  pltpu.make_async_copy(ones, accg.at[idxb.at[0]], sems).wait()

  @pl.when(w == 13)
  def _():
    pltpu.sync_copy(batch_hbm.at[pl.ds(N_CH_FULL * CH, N_TAIL)], idx32)
    pltpu.sync_copy(ones32, accg.at[idx32], add=True)

  plsc.subcore_barrier()

  @pl.loop(0, (NP_CH + NS - 1) // NS)
  def _(i):
    ch = s + i * NS

    @pl.when(ch < NP_CH)
    def _():
      pltpu.make_async_copy(accd.at[pl.ds(ch * CH, CH)],
                            cnt_hbm.at[c, pl.ds(ch * CH, CH)], semz).start()

  @pl.when(s < GP_CH)
  def _():
    pltpu.make_async_copy(accg.at[pl.ds(s * CH, CH)],
                          gcnt_hbm.at[c, pl.ds(s * CH, CH)], semz).start()

  @pl.loop(0, (NP_CH + NS - 1) // NS)
  def _(i):
    ch = s + i * NS

    @pl.when(ch < NP_CH)
    def _():
      pltpu.make_async_copy(accd.at[pl.ds(ch * CH, CH)],
                            cnt_hbm.at[c, pl.ds(ch * CH, CH)], semz).wait()

  @pl.when(s < GP_CH)
  def _():
    pltpu.make_async_copy(accg.at[pl.ds(s * CH, CH)],
                          gcnt_hbm.at[c, pl.ds(s * CH, CH)], semz).wait()


@functools.partial(
    pl.kernel,
    out_type=jax.ShapeDtypeStruct((NC, N_PAD, H), jnp.float32),
    mesh=_mesh,
    compiler_params=_sc_params,
    scratch_types=[
        pltpu.VMEM_SHARED((N_PAD, H), jnp.float32),
        pltpu.VMEM((2, KG, CH), jnp.int32),
        pltpu.VMEM((2, KG, CH), jnp.int32),
        pltpu.VMEM((KG, CH, H), jnp.float32),
        pltpu.VMEM((CH, H), jnp.float32),
        pltpu.SemaphoreType.DMA,
        pltpu.SemaphoreType.DMA,
        pltpu.SemaphoreType.DMA((KG,)),
        pltpu.SemaphoreType.DMA,
        pltpu.SemaphoreType.DMA,
    ],
)
def _sc_scatter(u_hbm, e_hbm, out_hbm,
                acc, isbuf, idbuf, rows, zrows,
                semis, semid, semg, sems, semz):
  c = lax.axis_index("c")
  s = lax.axis_index("s")
  w = s * NC + c
  ng = jnp.where(w < NG_REM, NG_MAX, NG_MAX - 1)

  _zero_rows(zrows)

  pltpu.make_async_copy(e_hbm.at[0, pl.ds(w * KG, KG)], isbuf.at[0],
                        semis).start()
  pltpu.make_async_copy(e_hbm.at[1, pl.ds(w * KG, KG)], idbuf.at[0],
                        semid).start()

  @pl.loop(0, (NP_CH + NS - 1) // NS)
  def _(i):
    ch = s + i * NS

    @pl.when(ch < NP_CH)
    def _():
      pltpu.make_async_copy(zrows, acc.at[pl.ds(ch * CH, CH)], semz).start()

  @pl.loop(0, (NP_CH + NS - 1) // NS)
  def _(i):
    ch = s + i * NS

    @pl.when(ch < NP_CH)
    def _():
      pltpu.make_async_copy(zrows, acc.at[pl.ds(ch * CH, CH)], semz).wait()

  plsc.subcore_barrier()

  @pl.loop(0, NG_MAX)
  def _(i):
    @pl.when(i < ng)
    def _():
      g = w + i * NC * NS
      b = i % 2
      pltpu.make_async_copy(e_hbm.at[0, pl.ds(g * KG, KG)], isbuf.at[b],
                            semis).wait()
      pltpu.make_async_copy(e_hbm.at[1, pl.ds(g * KG, KG)], idbuf.at[b],
                            semid).wait()

      @pl.when(i > 0)
      def _():
        for j in range(KG):
          pltpu.make_async_copy(rows.at[j], acc.at[idbuf.at[1 - b, j]],
                                sems).wait()

      @pl.when(i + 1 < ng)
      def _():
        g2 = w + (i + 1) * NC * NS
        pltpu.make_async_copy(e_hbm.at[0, pl.ds(g2 * KG, KG)],
                              isbuf.at[1 - b], semis).start()
        pltpu.make_async_copy(e_hbm.at[1, pl.ds(g2 * KG, KG)],
                              idbuf.at[1 - b], semid).start()

      for j in range(KG):
        pltpu.make_async_copy(u_hbm.at[isbuf.at[b, j]], rows.at[j],
                              semg.at[j]).start()
      for j in range(KG):
        pltpu.make_async_copy(u_hbm.at[isbuf.at[b, j]], rows.at[j],
                              semg.at[j]).wait()
        pltpu.make_async_copy(rows.at[j], acc.at[idbuf.at[b, j]],
                              sems).start(add=True)

  for j in range(KG):
    pltpu.make_async_copy(rows.at[j], acc.at[idbuf.at[0, j]], sems).wait()

  plsc.subcore_barrier()

  @pl.loop(0, (NP_CH + NS - 1) // NS)
  def _(i):
    ch = s + i * NS

    @pl.when(ch < NP_CH)
    def _():
      pltpu.make_async_copy(acc.at[pl.ds(ch * CH, CH)],
                            out_hbm.at[c, pl.ds(ch * CH, CH)], semz).start()

  @pl.loop(0, (NP_CH + NS - 1) // NS)
  def _(i):
    ch = s + i * NS

    @pl.when(ch < NP_CH)
    def _():
      pltpu.make_async_copy(acc.at[pl.ds(ch * CH, CH)],
                            out_hbm.at[c, pl.ds(ch * CH, CH)], semz).wait()


@functools.partial(
    pl.kernel,
    out_type=jax.ShapeDtypeStruct((NC, G_PAD, H), jnp.float32),
    mesh=_mesh,
    compiler_params=_sc_params,
    scratch_types=[
        pltpu.VMEM_SHARED((G_PAD, H), jnp.float32),
        pltpu.VMEM((2, CH), jnp.int32),
        pltpu.VMEM((2, CH, H), jnp.float32),
        pltpu.VMEM((CH, H), jnp.float32),
        pltpu.VMEM((N_TAIL,), jnp.int32),
        pltpu.VMEM((N_TAIL, H), jnp.float32),
        pltpu.SemaphoreType.DMA,
        pltpu.SemaphoreType.DMA,
        pltpu.SemaphoreType.DMA,
    ],
)
def _sc_pool(h_hbm, batch_hbm, out_hbm,
             accp, idxb, rows, zrows, idx32, rows32, semi, semr, sems):
  c = lax.axis_index("c")
  s = lax.axis_index("s")
  w = s * NC + c

  _zero_rows(zrows)

  pltpu.make_async_copy(batch_hbm.at[pl.ds(w * CH, CH)], idxb.at[0],
                        semi).start()
  pltpu.make_async_copy(h_hbm.at[pl.ds(w * CH, CH)], rows.at[0],
                        semr).start()

  @pl.when(s < GP_CH)
  def _():
    pltpu.sync_copy(zrows, accp.at[pl.ds(s * CH, CH)])

  plsc.subcore_barrier()

  @pl.loop(0, (N_CH_FULL + NC * NS - 1) // (NC * NS))
  def _(i):
    ch = w + i * NC * NS

    @pl.when(ch < N_CH_FULL)
    def _():
      b = i % 2
      pltpu.make_async_copy(batch_hbm.at[pl.ds(ch * CH, CH)], idxb.at[b],
                            semi).wait()
      pltpu.make_async_copy(h_hbm.at[pl.ds(ch * CH, CH)], rows.at[b],
                            semr).wait()

      @pl.when(i > 0)
      def _():
        pltpu.make_async_copy(rows.at[1 - b], accp.at[idxb.at[1 - b]],
                              sems).wait()

      ch2 = ch + NC * NS

      @pl.when(ch2 < N_CH_FULL)
      def _():
        pltpu.make_async_copy(batch_hbm.at[pl.ds(ch2 * CH, CH)],
                              idxb.at[1 - b], semi).start()
        pltpu.make_async_copy(h_hbm.at[pl.ds(ch2 * CH, CH)],
                              rows.at[1 - b], semr).start()

      pltpu.make_async_copy(rows.at[b], accp.at[idxb.at[b]],
                            sems).start(add=True)

  pltpu.make_async_copy(rows.at[0], accp.at[idxb.at[0]], sems).wait()

  @pl.when(w == 13)
  def _():
    pltpu.sync_copy(batch_hbm.at[pl.ds(N_CH_FULL * CH, N_TAIL)], idx32)
    pltpu.sync_copy(h_hbm.at[pl.ds(N_CH_FULL * CH, N_TAIL)], rows32)
    pltpu.sync_copy(rows32, accp.at[idx32], add=True)

  plsc.subcore_barrier()

  @pl.when(s < GP_CH)
  def _():
    pltpu.sync_copy(accp.at[pl.ds(s * CH, CH)],
                    out_hbm.at[c, pl.ds(s * CH, CH)])


_BN = 8192
_BNV = _BN // 8
_NV = N_PAD // 8
_GRID = (N_PAD + _BN - 1) // _BN


def _tc_a_body(xv_ref, cnt_ref, w_ref, u_ref, d_ref):
  dv = lax.rsqrt(cnt_ref[0] + cnt_ref[1] + 1.0)
  y = jnp.dot(xv_ref[...], w_ref[...], preferred_element_type=jnp.float32)
  u_ref[...] = y * dv
  d_ref[...] = dv


def _tc_a(xv, cv, W0k):
  return pl.pallas_call(
      _tc_a_body,
      grid=(_GRID,),
      in_specs=[
          pl.BlockSpec((_BNV, 8 * D_IN), lambda i: (i, 0)),
          pl.BlockSpec((NC, _BNV, 128), lambda i: (0, i, 0)),
          pl.BlockSpec((8 * D_IN, 128), lambda i: (0, 0)),
      ],
      out_specs=[
          pl.BlockSpec((_BNV, 128), lambda i: (i, 0)),
          pl.BlockSpec((_BNV, 128), lambda i: (i, 0)),
      ],
      out_shape=[
          jax.ShapeDtypeStruct((_NV, 128), jnp.float32),
          jax.ShapeDtypeStruct((_NV, 128), jnp.float32),
      ],
  )(xv, cv, W0k)


def _tc_b_body(s_ref, u_ref, d_ref, b_ref, w_ref, u1_ref):
  h1 = (s_ref[0] + s_ref[1] + u_ref[...]) * d_ref[...] + b_ref[...]
  h1 = jnp.maximum(h1, 0.0)
  u1_ref[...] = jnp.dot(h1, w_ref[...],
                        preferred_element_type=jnp.float32) * d_ref[...]


def _tc_b(s2v, u0v, dv, b0v, W1k):
  return pl.pallas_call(
      _tc_b_body,
      grid=(_GRID,),
      in_specs=[
          pl.BlockSpec((NC, _BNV, 128), lambda i: (0, i, 0)),
          pl.BlockSpec((_BNV, 128), lambda i: (i, 0)),
          pl.BlockSpec((_BNV, 128), lambda i: (i, 0)),
          pl.BlockSpec((1, 128), lambda i: (0, 0)),
          pl.BlockSpec((128, 128), lambda i: (0, 0)),
      ],
      out_specs=pl.BlockSpec((_BNV, 128), lambda i: (i, 0)),
      out_shape=jax.ShapeDtypeStruct((_NV, 128), jnp.float32),
  )(s2v, u0v, dv, b0v, W1k)


def _tc_c_body(s_ref, u_ref, d_ref, b_ref, h_ref):
  h_ref[...] = (s_ref[0] + s_ref[1] + u_ref[...]) * d_ref[...] \
      + b_ref[...]


def _tc_c(s2v, u1v, dv, b1v):
  return pl.pallas_call(
      _tc_c_body,
      grid=(_GRID,),
      in_specs=[
          pl.BlockSpec((NC, _BNV, 128), lambda i: (0, i, 0)),
          pl.BlockSpec((_BNV, 128), lambda i: (i, 0)),
          pl.BlockSpec((_BNV, 128), lambda i: (i, 0)),
          pl.BlockSpec((1, 128), lambda i: (0, 0)),
      ],
      out_specs=pl.BlockSpec((_BNV, 128), lambda i: (i, 0)),
      out_shape=jax.ShapeDtypeStruct((_NV, 128), jnp.float32),
  )(s2v, u1v, dv, b1v)


def _tc_d_body(p_ref, g_ref, wl_ref, bl_ref, o_ref):
  cnt = jnp.maximum(g_ref[0] + g_ref[1], 1.0)
  p = (p_ref[0] + p_ref[1]) / cnt
  o_ref[...] = jax.nn.sigmoid(
      jnp.dot(p, wl_ref[...], preferred_element_type=jnp.float32)
      + bl_ref[...])


def _tc_d(p2v, gv, Wlk, bl):
  return pl.pallas_call(
      _tc_d_body,
      out_shape=jax.ShapeDtypeStruct((G_PAD // 8, 8), jnp.float32),
  )(p2v, gv, Wlk, bl)


def kernel(x, edge_index, batch, W0, b0, W1, b1, Wl, bl):
  e3 = edge_index.reshape(2, E_CH, CH)

  eye8 = jnp.eye(8, dtype=jnp.float32)
  W0k = jnp.kron(eye8, W0)
  W1k = jnp.kron(eye8, W1)
  Wlk = jnp.kron(eye8, Wl)
  b0v = jnp.tile(b0, 8).reshape(1, 128)
  b1v = jnp.tile(b1, 8).reshape(1, 128)

  xv = jnp.pad(x, ((0, N_PAD - N), (0, 0))).reshape(_NV, 8 * D_IN)

  cnt16, gcnt16 = _sc_count(e3, batch)

  cv = cnt16.reshape(NC, _NV, 128)
  gv = gcnt16.reshape(NC, G_PAD // 8, 128)

  u0v, dv = _tc_a(xv, cv, W0k)

  s0 = _sc_scatter(u0v.reshape(N_PAD, H), e3)
  u1v = _tc_b(s0.reshape(NC, _NV, 128), u0v, dv, b0v, W1k)

  s1 = _sc_scatter(u1v.reshape(N_PAD, H), e3)
  h2v = _tc_c(s1.reshape(NC, _NV, 128), u1v, dv, b1v)

  p2 = _sc_pool(h2v.reshape(N_PAD, H), batch)

  out = _tc_d(p2.reshape(NC, G_PAD // 8, 128), gv, Wlk, bl.reshape(1, 1))
  return out.reshape(G_PAD, 1)[:G]
